# Initial kernel scaffold; baseline (speedup 1.0000x reference)
import jax
import jax.numpy as jnp
from jax.experimental import pallas as pl


def kernel(x, edge_index, edge_attr, pe_w1, pe_b1, pe_w2, pe_b2, c1_lin_w, c1_lin_b, c1_kw, c1_kb, c1_sw, c1_sb, c2_lin_w, c2_lin_b, c2_kw, c2_kb, c2_sw, c2_sb):
    raise NotImplementedError("write your pallas kernel here")


# speedup vs baseline: 1.0207x; 1.0207x over previous
import functools

import jax
import jax.numpy as jnp
from jax.experimental import pallas as pl
from jax.experimental.pallas import tpu as pltpu

N = 10000
E = 320000
IN_CH = 128
HID = 128
OUT_CH = 1
D_EDGE = 16
L = 2

_BN = 1000


def _pe_lin_body(deg_ref, x_ref, w1_ref, b1_ref, w2_ref, b2_ref, lw_ref, lb_ref, o_ref):
    pef = jnp.log1p(deg_ref[...])
    a = jax.nn.relu(pef * w1_ref[...] + b1_ref[...])
    pe = a @ w2_ref[...] + b2_ref[...]
    h0 = x_ref[...] + pe
    o_ref[...] = h0 @ lw_ref[...] + lb_ref[...]


def _pe_lin(deg, x, w1, b1, w2, b2, lw, lb):
    grid = (N // _BN,)
    return pl.pallas_call(
        _pe_lin_body,
        grid=grid,
        in_specs=[
            pl.BlockSpec((_BN, 1), lambda i: (i, 0)),
            pl.BlockSpec((_BN, IN_CH), lambda i: (i, 0)),
            pl.BlockSpec((1, HID), lambda i: (0, 0)),
            pl.BlockSpec((1, HID), lambda i: (0, 0)),
            pl.BlockSpec((HID, IN_CH), lambda i: (0, 0)),
            pl.BlockSpec((1, IN_CH), lambda i: (0, 0)),
            pl.BlockSpec((IN_CH, HID), lambda i: (0, 0)),
            pl.BlockSpec((1, HID), lambda i: (0, 0)),
        ],
        out_specs=pl.BlockSpec((_BN, HID), lambda i: (i, 0)),
        out_shape=jax.ShapeDtypeStruct((N, HID), jnp.float32),
    )(deg, x, w1, b1, w2, b2, lw, lb)


def _update_body(h_ref, agg_ref, dinv_ref, sw_ref, sb_ref, o_ref):
    v = h_ref[...] @ sw_ref[...] + sb_ref[...] + agg_ref[...] * dinv_ref[...]
    o_ref[...] = jax.nn.relu(v)


def _update(h, agg, dinv, sw, sb):
    c = h.shape[1]
    grid = (N // _BN,)
    return pl.pallas_call(
        _update_body,
        grid=grid,
        in_specs=[
            pl.BlockSpec((_BN, c), lambda i: (i, 0)),
            pl.BlockSpec((_BN, c), lambda i: (i, 0)),
            pl.BlockSpec((_BN, 1), lambda i: (i, 0)),
            pl.BlockSpec((c, c), lambda i: (0, 0)),
            pl.BlockSpec((1, c), lambda i: (0, 0)),
        ],
        out_specs=pl.BlockSpec((_BN, c), lambda i: (i, 0)),
        out_shape=jax.ShapeDtypeStruct((N, c), jnp.float32),
    )(h, agg, dinv, sw, sb)


def kernel(x, edge_index, edge_attr, pe_w1, pe_b1, pe_w2, pe_b2,
           c1_lin_w, c1_lin_b, c1_kw, c1_kb, c1_sw, c1_sb,
           c2_lin_w, c2_lin_b, c2_kw, c2_kb, c2_sw, c2_sb):
    src = edge_index[0]
    dst = edge_index[1]
    deg = jax.ops.segment_sum(jnp.ones((E,), jnp.float32), dst, num_segments=N)
    deg = jnp.clip(deg, 1.0, None)
    dinv = (1.0 / deg)[:, None]

    h = _pe_lin(deg[:, None], x, pe_w1, pe_b1[None, :], pe_w2, pe_b2[None, :],
                c1_lin_w, c1_lin_b[None, :])
    for l in range(L):
        ker = edge_attr @ c1_kw[l] + c1_kb[l]
        msg = ker * jnp.take(h, src, axis=0)
        agg = jax.ops.segment_sum(msg, dst, num_segments=N)
        h = _update(h, agg, dinv, c1_sw[l], c1_sb[l][None, :])

    h = h @ c2_lin_w + c2_lin_b
    for l in range(L):
        ker = edge_attr @ c2_kw[l] + c2_kb[l]
        msg = ker * jnp.take(h, src, axis=0)
        agg = jax.ops.segment_sum(msg, dst, num_segments=N)
        h = jax.nn.relu(h * c2_sw[l][0, 0] + c2_sb[l][None, :] + agg * dinv)
    return h


# speedup vs baseline: 3.9990x; 3.9178x over previous
import functools

import jax
import jax.numpy as jnp
from jax import lax
from jax.experimental import pallas as pl
from jax.experimental.pallas import tpu as pltpu
from jax.experimental.pallas import tpu_sc as plsc

N = 10000
E = 320000
IN_CH = 128
HID = 128
OUT_CH = 1
D_EDGE = 16
L = 2

NPAD = 10240
NROW = NPAD // 128
NW = 32
CH = 128
NCHUNK = E // CH
FULL_ROUNDS = NCHUNK // NW
TAIL = NCHUNK - FULL_ROUNDS * NW
NR = 10112
RPT = NR // 16
FPT = NPAD // 16

_BN = 1000


def _sc_mesh():
    return plsc.VectorSubcoreMesh(core_axis_name="c", subcore_axis_name="s",
                                  num_cores=2, num_subcores=16)


def _deg_body(dst_hbm, zeros_hbm, out_hbm, dstbuf, ones_v, deg_sh, dsem):
    c = lax.axis_index("c")
    s = lax.axis_index("s")
    wid = s * 2 + c

    def _ones(i, _):
        ones_v[pl.ds(i * 16, 16)] = jnp.full((16,), 1.0, jnp.float32)
        return 0
    lax.fori_loop(0, CH // 16, _ones, 0)

    pltpu.sync_copy(zeros_hbm.at[pl.ds(s * FPT, FPT)], deg_sh.at[pl.ds(s * FPT, FPT)])
    plsc.subcore_barrier()

    def _chunk(j):
        e0 = j * CH
        pltpu.sync_copy(dst_hbm.at[pl.ds(e0, CH)], dstbuf)
        pltpu.sync_copy(ones_v, deg_sh.at[dstbuf], add=True)

    def _round(r, _):
        _chunk(wid + r * NW)
        return 0
    lax.fori_loop(0, FULL_ROUNDS, _round, 0)

    @pl.when(wid < TAIL)
    def _():
        _chunk(FULL_ROUNDS * NW + wid)

    plsc.subcore_barrier()
    pltpu.sync_copy(deg_sh.at[pl.ds(s * FPT, FPT)], out_hbm.at[c, pl.ds(s * FPT, FPT)])


def _sc_deg(dst, zeros_flat):
    f = pl.kernel(
        _deg_body,
        out_type=jax.ShapeDtypeStruct((2, NPAD), jnp.float32),
        mesh=_sc_mesh(),
        scratch_types=[
            pltpu.VMEM((CH,), jnp.int32),
            pltpu.VMEM((CH,), jnp.float32),
            pltpu.VMEM_SHARED((NPAD,), jnp.float32),
            pltpu.SemaphoreType.DMA,
        ],
    )
    return f(dst, zeros_flat)


def _agg_body(h_hbm, ker_hbm, src_hbm, dst_hbm, zeros_hbm, out_hbm,
              srcbuf, dstbuf, hs, kerb, msg, agg_sh, gsem):
    c = lax.axis_index("c")
    s = lax.axis_index("s")
    wid = s * 2 + c

    pltpu.sync_copy(zeros_hbm.at[pl.ds(s * RPT, RPT)], agg_sh.at[pl.ds(s * RPT, RPT)])
    plsc.subcore_barrier()

    def _chunk(j):
        e0 = j * CH
        pltpu.sync_copy(src_hbm.at[pl.ds(e0, CH)], srcbuf)
        pltpu.sync_copy(dst_hbm.at[pl.ds(e0, CH)], dstbuf)
        pltpu.async_copy(h_hbm.at[srcbuf], hs, gsem).wait()
        pltpu.sync_copy(ker_hbm.at[pl.ds(e0, CH), :], kerb)

        def _mul(i, _):
            for q in range(HID // 16):
                sl = pl.ds(q * 16, 16)
                msg[i, sl] = hs[i, sl] * kerb[i, sl]
            return 0
        lax.fori_loop(0, CH, _mul, 0)
        pltpu.sync_copy(msg, agg_sh.at[dstbuf], add=True)

    def _round(r, _):
        _chunk(wid + r * NW)
        return 0
    lax.fori_loop(0, FULL_ROUNDS, _round, 0)

    @pl.when(wid < TAIL)
    def _():
        _chunk(FULL_ROUNDS * NW + wid)

    plsc.subcore_barrier()
    pltpu.sync_copy(agg_sh.at[pl.ds(s * RPT, RPT)], out_hbm.at[c, pl.ds(s * RPT, RPT)])


def _sc_agg(h, ker, src, dst, zeros_h):
    f = pl.kernel(
        _agg_body,
        out_type=jax.ShapeDtypeStruct((2, NR, HID), jnp.float32),
        mesh=_sc_mesh(),
        scratch_types=[
            pltpu.VMEM((CH,), jnp.int32),
            pltpu.VMEM((CH,), jnp.int32),
            pltpu.VMEM((CH, HID), jnp.float32),
            pltpu.VMEM((CH, HID), jnp.float32),
            pltpu.VMEM((CH, HID), jnp.float32),
            pltpu.VMEM_SHARED((NR, HID), jnp.float32),
            pltpu.SemaphoreType.DMA,
        ],
    )
    return f(h, ker, src, dst, zeros_h)


def _agg2_body(g_hbm, ker_hbm, src_hbm, dst_hbm, zeros_hbm, out_hbm,
               srcbuf, dstbuf, hsbuf, kerb, msg, agg_sh, gsem):
    c = lax.axis_index("c")
    s = lax.axis_index("s")
    wid = s * 2 + c

    pltpu.sync_copy(zeros_hbm.at[pl.ds(s * FPT, FPT)], agg_sh.at[pl.ds(s * FPT, FPT)])
    plsc.subcore_barrier()

    def _chunk(j):
        e0 = j * CH
        pltpu.sync_copy(src_hbm.at[pl.ds(e0, CH)], srcbuf)
        pltpu.sync_copy(dst_hbm.at[pl.ds(e0, CH)], dstbuf)
        pltpu.async_copy(g_hbm.at[srcbuf], hsbuf, gsem).wait()
        pltpu.sync_copy(ker_hbm.at[pl.ds(e0, CH)], kerb)

        def _mul(i, _):
            sl = pl.ds(i * 16, 16)
            msg[sl] = hsbuf[sl] * kerb[sl]
            return 0
        lax.fori_loop(0, CH // 16, _mul, 0)
        pltpu.sync_copy(msg, agg_sh.at[dstbuf], add=True)

    def _round(r, _):
        _chunk(wid + r * NW)
        return 0
    lax.fori_loop(0, FULL_ROUNDS, _round, 0)

    @pl.when(wid < TAIL)
    def _():
        _chunk(FULL_ROUNDS * NW + wid)

    plsc.subcore_barrier()
    pltpu.sync_copy(agg_sh.at[pl.ds(s * FPT, FPT)], out_hbm.at[c, pl.ds(s * FPT, FPT)])


def _sc_agg2(g_flat, ker2, src, dst, zeros_flat):
    f = pl.kernel(
        _agg2_body,
        out_type=jax.ShapeDtypeStruct((2, NPAD), jnp.float32),
        mesh=_sc_mesh(),
        scratch_types=[
            pltpu.VMEM((CH,), jnp.int32),
            pltpu.VMEM((CH,), jnp.int32),
            pltpu.VMEM((CH,), jnp.float32),
            pltpu.VMEM((CH,), jnp.float32),
            pltpu.VMEM((CH,), jnp.float32),
            pltpu.VMEM_SHARED((NPAD,), jnp.float32),
            pltpu.SemaphoreType.DMA,
        ],
    )
    return f(g_flat, ker2, src, dst, zeros_flat)


def _ker1_body(ea_ref, kw_ref, kb_ref, o_ref):
    o_ref[...] = (jnp.dot(ea_ref[...], kw_ref[0], preferred_element_type=jnp.float32)
                  + kb_ref[0])[None]


def _tc_ker1(edge_attr, kw, kb):
    be = 4000
    return pl.pallas_call(
        _ker1_body,
        grid=(L, E // be),
        in_specs=[
            pl.BlockSpec((be, D_EDGE), lambda l, i: (i, 0)),
            pl.BlockSpec((1, D_EDGE, HID), lambda l, i: (l, 0, 0)),
            pl.BlockSpec((1, 1, HID), lambda l, i: (l, 0, 0)),
        ],
        out_specs=pl.BlockSpec((1, be, HID), lambda l, i: (l, i, 0)),
        out_shape=jax.ShapeDtypeStruct((L, E, HID), jnp.float32),
    )(edge_attr, kw, kb)


def _ker2_body(ea_ref, kw_ref, kb_ref, o_ref):
    o_ref[...] = jnp.dot(ea_ref[...], kw_ref[...], preferred_element_type=jnp.float32) + kb_ref[...]


def _tc_ker2(edge_attr, kw2, kb2):
    be = 8000
    return pl.pallas_call(
        _ker2_body,
        grid=(E // be,),
        in_specs=[
            pl.BlockSpec((be, D_EDGE), lambda i: (i, 0)),
            pl.BlockSpec((D_EDGE, L), lambda i: (0, 0)),
            pl.BlockSpec((1, L), lambda i: (0, 0)),
        ],
        out_specs=pl.BlockSpec((be, L), lambda i: (i, 0)),
        out_shape=jax.ShapeDtypeStruct((E, L), jnp.float32),
    )(edge_attr, kw2, kb2)


def _pe_body(deg_ref, x_ref, w1_ref, b1_ref, w2_ref, b2_ref, lw_ref, lb_ref,
             h_ref, dinv_ref):
    deg = jnp.clip(deg_ref[...], 1.0, None)
    dinv_ref[...] = 1.0 / deg
    pef = jnp.log1p(deg)
    a = jax.nn.relu(pef * w1_ref[...] + b1_ref[...])
    pe = jnp.dot(a, w2_ref[...], preferred_element_type=jnp.float32) + b2_ref[...]
    h0 = x_ref[...] + pe
    h_ref[...] = jnp.dot(h0, lw_ref[...], preferred_element_type=jnp.float32) + lb_ref[...]


def _tc_pe(deg_col, x, w1, b1, w2, b2, lw, lb):
    return pl.pallas_call(
        _pe_body,
        grid=(N // _BN,),
        in_specs=[
            pl.BlockSpec((_BN, 1), lambda i: (i, 0)),
            pl.BlockSpec((_BN, IN_CH), lambda i: (i, 0)),
            pl.BlockSpec((1, HID), lambda i: (0, 0)),
            pl.BlockSpec((1, HID), lambda i: (0, 0)),
            pl.BlockSpec((HID, IN_CH), lambda i: (0, 0)),
            pl.BlockSpec((1, IN_CH), lambda i: (0, 0)),
            pl.BlockSpec((IN_CH, HID), lambda i: (0, 0)),
            pl.BlockSpec((1, HID), lambda i: (0, 0)),
        ],
        out_specs=[
            pl.BlockSpec((_BN, HID), lambda i: (i, 0)),
            pl.BlockSpec((_BN, 1), lambda i: (i, 0)),
        ],
        out_shape=[
            jax.ShapeDtypeStruct((N, HID), jnp.float32),
            jax.ShapeDtypeStruct((N, 1), jnp.float32),
        ],
    )(deg_col, x, w1, b1, w2, b2, lw, lb)


def _upd_body(h_ref, aggp_ref, dinv_ref, sw_ref, sb_ref, o_ref):
    agg = (aggp_ref[0] + aggp_ref[1]) * dinv_ref[...]
    o_ref[...] = jax.nn.relu(
        jnp.dot(h_ref[...], sw_ref[...], preferred_element_type=jnp.float32)
        + sb_ref[...] + agg)


def _tc_update(h, aggp, dinv, sw, sb):
    return pl.pallas_call(
        _upd_body,
        grid=(N // _BN,),
        in_specs=[
            pl.BlockSpec((_BN, HID), lambda i: (i, 0)),
            pl.BlockSpec((2, _BN, HID), lambda i: (0, i, 0)),
            pl.BlockSpec((_BN, 1), lambda i: (i, 0)),
            pl.BlockSpec((HID, HID), lambda i: (0, 0)),
            pl.BlockSpec((1, HID), lambda i: (0, 0)),
        ],
        out_specs=pl.BlockSpec((_BN, HID), lambda i: (i, 0)),
        out_shape=jax.ShapeDtypeStruct((N, HID), jnp.float32),
    )(h, aggp, dinv, sw, sb)


def _updlin_body(h_ref, aggp_ref, dinv_ref, sw_ref, sb_ref, lw_ref, lb_ref, o_ref):
    agg = (aggp_ref[0] + aggp_ref[1]) * dinv_ref[...]
    h1 = jax.nn.relu(
        jnp.dot(h_ref[...], sw_ref[...], preferred_element_type=jnp.float32)
        + sb_ref[...] + agg)
    o_ref[...] = jnp.dot(h1, lw_ref[...], preferred_element_type=jnp.float32) + lb_ref[...]


def _tc_update_lin(h, aggp, dinv, sw, sb, lw, lb):
    return pl.pallas_call(
        _updlin_body,
        grid=(N // _BN,),
        in_specs=[
            pl.BlockSpec((_BN, HID), lambda i: (i, 0)),
            pl.BlockSpec((2, _BN, HID), lambda i: (0, i, 0)),
            pl.BlockSpec((_BN, 1), lambda i: (i, 0)),
            pl.BlockSpec((HID, HID), lambda i: (0, 0)),
            pl.BlockSpec((1, HID), lambda i: (0, 0)),
            pl.BlockSpec((HID, OUT_CH), lambda i: (0, 0)),
            pl.BlockSpec((1, OUT_CH), lambda i: (0, 0)),
        ],
        out_specs=pl.BlockSpec((_BN, OUT_CH), lambda i: (i, 0)),
        out_shape=jax.ShapeDtypeStruct((N, OUT_CH), jnp.float32),
    )(h, aggp, dinv, sw, sb, lw, lb)


def _upd2_body(g_ref, aggp_ref, dinvf_ref, sw_ref, sb_ref, o_ref):
    agg = (aggp_ref[0] + aggp_ref[1]) * dinvf_ref[...]
    o_ref[...] = jax.nn.relu(g_ref[...] * sw_ref[0, 0] + sb_ref[0, 0] + agg)


def _tc_update2(g_flat2d, aggp_flat, dinv_flat2d, sw_s, sb_s):
    return pl.pallas_call(
        _upd2_body,
        grid=(1,),
        in_specs=[
            pl.BlockSpec((NROW, 128), lambda i: (0, 0)),
            pl.BlockSpec((2, NROW, 128), lambda i: (0, 0, 0)),
            pl.BlockSpec((NROW, 128), lambda i: (0, 0)),
            pl.BlockSpec((1, 1), lambda i: (0, 0)),
            pl.BlockSpec((1, 1), lambda i: (0, 0)),
        ],
        out_specs=pl.BlockSpec((NROW, 128), lambda i: (0, 0)),
        out_shape=jax.ShapeDtypeStruct((NROW, 128), jnp.float32),
    )(g_flat2d, aggp_flat, dinv_flat2d, sw_s, sb_s)


def kernel(x, edge_index, edge_attr, pe_w1, pe_b1, pe_w2, pe_b2,
           c1_lin_w, c1_lin_b, c1_kw, c1_kb, c1_sw, c1_sb,
           c2_lin_w, c2_lin_b, c2_kw, c2_kb, c2_sw, c2_sb):
    src = edge_index[0]
    dst = edge_index[1]
    zeros_flat = jnp.zeros((NPAD,), jnp.float32)
    zeros_h = jnp.zeros((NR, HID), jnp.float32)

    degp = _sc_deg(dst, zeros_flat)
    deg_col = (degp[0] + degp[1])[:N, None]

    ker1 = _tc_ker1(edge_attr, c1_kw, c1_kb[:, None, :])
    ker2 = _tc_ker2(edge_attr, jnp.transpose(c2_kw, (1, 2, 0)).reshape(D_EDGE, L),
                    c2_kb.reshape(1, L))

    h, dinv = _tc_pe(deg_col, x, pe_w1, pe_b1[None, :], pe_w2, pe_b2[None, :],
                     c1_lin_w, c1_lin_b[None, :])

    for l in range(L):
        aggp = _sc_agg(h, ker1[l], src, dst, zeros_h)
        if l < L - 1:
            h = _tc_update(h, aggp, dinv, c1_sw[l], c1_sb[l][None, :])
        else:
            g = _tc_update_lin(h, aggp, dinv, c1_sw[l], c1_sb[l][None, :],
                               c2_lin_w, c2_lin_b[None, :])

    dinv_flat2d = jnp.pad(dinv[:, 0], (0, NPAD - N)).reshape(NROW, 128)
    g_flat = jnp.pad(g[:, 0], (0, NPAD - N))
    for l in range(L):
        aggp2 = _sc_agg2(g_flat, ker2[:, l], src, dst, zeros_flat)
        g2d = _tc_update2(g_flat.reshape(NROW, 128),
                          aggp2.reshape(2, NROW, 128), dinv_flat2d,
                          c2_sw[l], c2_sb[l].reshape(1, 1))
        g_flat = g2d.reshape(NPAD)

    return g_flat[:N, None]


# speedup vs baseline: 4.2024x; 1.0509x over previous
import functools

import jax
import jax.numpy as jnp
from jax import lax
from jax.experimental import pallas as pl
from jax.experimental.pallas import tpu as pltpu
from jax.experimental.pallas import tpu_sc as plsc

N = 10000
E = 320000
IN_CH = 128
HID = 128
OUT_CH = 1
D_EDGE = 16
L = 2

NPAD = 10240
NROW = NPAD // 128
NW = 32
CH = 128
NCHUNK = E // CH
FULL_ROUNDS = NCHUNK // NW
TAIL = NCHUNK - FULL_ROUNDS * NW
CH1 = 64
NCHUNK1 = E // CH1
ROUNDS1 = NCHUNK1 // NW
TAIL1 = NCHUNK1 - ROUNDS1 * NW
NR = 10112
RPT = NR // 16
EPT = E // NW
FPT = NPAD // 16

_BN = 1000


def _sc_mesh():
    return plsc.VectorSubcoreMesh(core_axis_name="c", subcore_axis_name="s",
                                  num_cores=2, num_subcores=16)


def _deg_body(dst_hbm, zeros_hbm, out_hbm, dstb, ones_v, deg_sh, s_dst, s_s):
    c = lax.axis_index("c")
    s = lax.axis_index("s")
    wid = s * 2 + c
    cnt = jnp.where(wid < TAIL, FULL_ROUNDS + 1, FULL_ROUNDS)

    def ob(i, _):
        ones_v[pl.ds(i * 16, 16)] = jnp.full((16,), 1.0, jnp.float32)
        return 0
    lax.fori_loop(0, CH // 16, ob, 0)

    pltpu.sync_copy(zeros_hbm.at[pl.ds(s * FPT, FPT)], deg_sh.at[pl.ds(s * FPT, FPT)])
    plsc.subcore_barrier()

    def e0_of(r):
        return (wid + r * NW) * CH

    def fire_idx(r):
        b4 = lax.rem(r, 4)
        pltpu.async_copy(dst_hbm.at[pl.ds(e0_of(r), CH)], dstb.at[b4], s_dst.at[b4])

    fire_idx(0)
    @pl.when(cnt > 1)
    def _():
        fire_idx(1)

    def body(g, _):
        b2 = lax.rem(g, 2)
        b4 = lax.rem(g, 4)
        pltpu.make_async_copy(dst_hbm.at[pl.ds(0, CH)], dstb.at[b4], s_dst.at[b4]).wait()
        @pl.when(g >= 2)
        def _():
            pltpu.make_async_copy(ones_v, deg_sh.at[dstb.at[b4]], s_s.at[b2]).wait()
        pltpu.async_copy(ones_v, deg_sh.at[dstb.at[b4]], s_s.at[b2], add=True)
        @pl.when(g + 2 < cnt)
        def _():
            fire_idx(g + 2)
        return 0
    lax.fori_loop(0, cnt, body, 0)

    def drain(g):
        b2 = lax.rem(g, 2)
        b4 = lax.rem(g, 4)
        pltpu.make_async_copy(ones_v, deg_sh.at[dstb.at[b4]], s_s.at[b2]).wait()
    drain(cnt - 2)
    drain(cnt - 1)

    plsc.subcore_barrier()
    pltpu.sync_copy(deg_sh.at[pl.ds(s * FPT, FPT)], out_hbm.at[c, pl.ds(s * FPT, FPT)])


def _sc_deg(dst, zeros_flat):
    f = pl.kernel(
        _deg_body,
        out_type=jax.ShapeDtypeStruct((2, NPAD), jnp.float32),
        mesh=_sc_mesh(),
        scratch_types=[
            pltpu.VMEM((4, CH), jnp.int32),
            pltpu.VMEM((CH,), jnp.float32),
            pltpu.VMEM_SHARED((NPAD,), jnp.float32),
            pltpu.SemaphoreType.DMA((4,)),
            pltpu.SemaphoreType.DMA((2,)),
        ],
    )
    return f(dst, zeros_flat)


def _agg_body(h_hbm, ker_hbm, src_hbm, dst_hbm, zeros_hbm, out_hbm,
              srcb, dstb, hs, kerb, msg, agg_sh,
              s_src, s_dst, s_h, s_k, s_s):
    c = lax.axis_index("c")
    s = lax.axis_index("s")
    wid = s * 2 + c
    cnt = jnp.where(wid < TAIL1, ROUNDS1 + 1, ROUNDS1)

    pltpu.sync_copy(zeros_hbm.at[pl.ds(s * RPT, RPT)], agg_sh.at[pl.ds(s * RPT, RPT)])
    plsc.subcore_barrier()

    def e0_of(r):
        return (wid + r * NW) * CH1

    def fire_idx(r):
        b4 = lax.rem(r, 4)
        pltpu.async_copy(src_hbm.at[pl.ds(e0_of(r), CH1)], srcb.at[b4], s_src.at[b4])
        pltpu.async_copy(dst_hbm.at[pl.ds(e0_of(r), CH1)], dstb.at[b4], s_dst.at[b4])

    def fire_ker(r):
        b2 = lax.rem(r, 2)
        pltpu.async_copy(ker_hbm.at[pl.ds(e0_of(r), CH1)], kerb.at[b2], s_k.at[b2])

    def fire_gather(r):
        b4 = lax.rem(r, 4)
        b2 = lax.rem(r, 2)
        pltpu.make_async_copy(src_hbm.at[pl.ds(0, CH1)], srcb.at[b4], s_src.at[b4]).wait()
        pltpu.async_copy(h_hbm.at[srcb.at[b4]], hs.at[b2], s_h.at[b2])

    fire_idx(0)
    fire_ker(0)
    @pl.when(cnt > 1)
    def _():
        fire_idx(1)
        fire_ker(1)
    fire_gather(0)

    def body(g, _):
        b2 = lax.rem(g, 2)
        b4 = lax.rem(g, 4)

        @pl.when(g + 1 < cnt)
        def _():
            fire_gather(g + 1)

        pltpu.make_async_copy(h_hbm.at[srcb.at[b4]], hs.at[b2], s_h.at[b2]).wait()
        pltpu.make_async_copy(ker_hbm.at[pl.ds(0, CH1)], kerb.at[b2], s_k.at[b2]).wait()
        @pl.when(g >= 2)
        def _():
            pltpu.make_async_copy(msg.at[b2], agg_sh.at[dstb.at[b4]], s_s.at[b2]).wait()
        pltpu.make_async_copy(dst_hbm.at[pl.ds(0, CH1)], dstb.at[b4], s_dst.at[b4]).wait()

        def mul(i, _):
            for q in range(HID // 16):
                sl = pl.ds(q * 16, 16)
                msg[b2, i, sl] = hs[b2, i, sl] * kerb[b2, i, sl]
            return 0
        lax.fori_loop(0, CH1, mul, 0)

        pltpu.async_copy(msg.at[b2], agg_sh.at[dstb.at[b4]], s_s.at[b2], add=True)

        @pl.when(g + 2 < cnt)
        def _():
            fire_idx(g + 2)
            fire_ker(g + 2)
        return 0
    lax.fori_loop(0, cnt, body, 0)

    def drain(g):
        b2 = lax.rem(g, 2)
        b4 = lax.rem(g, 4)
        pltpu.make_async_copy(msg.at[b2], agg_sh.at[dstb.at[b4]], s_s.at[b2]).wait()
    drain(cnt - 2)
    drain(cnt - 1)

    plsc.subcore_barrier()
    pltpu.sync_copy(agg_sh.at[pl.ds(s * RPT, RPT)], out_hbm.at[c, pl.ds(s * RPT, RPT)])


def _sc_agg(h, ker, src, dst, zeros_h):
    f = pl.kernel(
        _agg_body,
        out_type=jax.ShapeDtypeStruct((2, NR, HID), jnp.float32),
        mesh=_sc_mesh(),
        scratch_types=[
            pltpu.VMEM((4, CH1), jnp.int32),
            pltpu.VMEM((4, CH1), jnp.int32),
            pltpu.VMEM((2, CH1, HID), jnp.float32),
            pltpu.VMEM((2, CH1, HID), jnp.float32),
            pltpu.VMEM((2, CH1, HID), jnp.float32),
            pltpu.VMEM_SHARED((NR, HID), jnp.float32),
            pltpu.SemaphoreType.DMA((4,)),
            pltpu.SemaphoreType.DMA((4,)),
            pltpu.SemaphoreType.DMA((2,)),
            pltpu.SemaphoreType.DMA((2,)),
            pltpu.SemaphoreType.DMA((2,)),
        ],
    )
    return f(h, ker, src, dst, zeros_h)


def _agg2_body(g_hbm, ker_hbm, src_hbm, dst_hbm, zeros_hbm, out_hbm,
               srcb, dstb, hsb, kerb, msgb, agg_sh,
               s_src, s_dst, s_h, s_k, s_s):
    c = lax.axis_index("c")
    s = lax.axis_index("s")
    wid = s * 2 + c
    cnt = jnp.where(wid < TAIL, FULL_ROUNDS + 1, FULL_ROUNDS)

    pltpu.sync_copy(zeros_hbm.at[pl.ds(s * FPT, FPT)], agg_sh.at[pl.ds(s * FPT, FPT)])
    plsc.subcore_barrier()

    def e0_of(r):
        return (wid + r * NW) * CH

    def fire_idx(r):
        b4 = lax.rem(r, 4)
        pltpu.async_copy(src_hbm.at[pl.ds(e0_of(r), CH)], srcb.at[b4], s_src.at[b4])
        pltpu.async_copy(dst_hbm.at[pl.ds(e0_of(r), CH)], dstb.at[b4], s_dst.at[b4])

    def fire_ker(r):
        b2 = lax.rem(r, 2)
        pltpu.async_copy(ker_hbm.at[pl.ds(e0_of(r), CH)], kerb.at[b2], s_k.at[b2])

    def fire_gather(r):
        b4 = lax.rem(r, 4)
        b2 = lax.rem(r, 2)
        pltpu.make_async_copy(src_hbm.at[pl.ds(0, CH)], srcb.at[b4], s_src.at[b4]).wait()
        pltpu.async_copy(g_hbm.at[srcb.at[b4]], hsb.at[b2], s_h.at[b2])

    fire_idx(0)
    fire_ker(0)
    @pl.when(cnt > 1)
    def _():
        fire_idx(1)
        fire_ker(1)
    fire_gather(0)

    def body(g, _):
        b2 = lax.rem(g, 2)
        b4 = lax.rem(g, 4)

        @pl.when(g + 1 < cnt)
        def _():
            fire_gather(g + 1)

        pltpu.make_async_copy(g_hbm.at[srcb.at[b4]], hsb.at[b2], s_h.at[b2]).wait()
        pltpu.make_async_copy(ker_hbm.at[pl.ds(0, CH)], kerb.at[b2], s_k.at[b2]).wait()
        @pl.when(g >= 2)
        def _():
            pltpu.make_async_copy(msgb.at[b2], agg_sh.at[dstb.at[b4]], s_s.at[b2]).wait()
        pltpu.make_async_copy(dst_hbm.at[pl.ds(0, CH)], dstb.at[b4], s_dst.at[b4]).wait()

        def mul(i, _):
            sl = pl.ds(i * 16, 16)
            msgb[b2, sl] = hsb[b2, sl] * kerb[b2, sl]
            return 0
        lax.fori_loop(0, CH // 16, mul, 0)

        pltpu.async_copy(msgb.at[b2], agg_sh.at[dstb.at[b4]], s_s.at[b2], add=True)

        @pl.when(g + 2 < cnt)
        def _():
            fire_idx(g + 2)
            fire_ker(g + 2)
        return 0
    lax.fori_loop(0, cnt, body, 0)

    def drain(g):
        b2 = lax.rem(g, 2)
        b4 = lax.rem(g, 4)
        pltpu.make_async_copy(msgb.at[b2], agg_sh.at[dstb.at[b4]], s_s.at[b2]).wait()
    drain(cnt - 2)
    drain(cnt - 1)

    plsc.subcore_barrier()
    pltpu.sync_copy(agg_sh.at[pl.ds(s * FPT, FPT)], out_hbm.at[c, pl.ds(s * FPT, FPT)])


def _sc_agg2(g_flat, ker2, src, dst, zeros_flat):
    f = pl.kernel(
        _agg2_body,
        out_type=jax.ShapeDtypeStruct((2, NPAD), jnp.float32),
        mesh=_sc_mesh(),
        scratch_types=[
            pltpu.VMEM((4, CH), jnp.int32),
            pltpu.VMEM((4, CH), jnp.int32),
            pltpu.VMEM((2, CH), jnp.float32),
            pltpu.VMEM((2, CH), jnp.float32),
            pltpu.VMEM((2, CH), jnp.float32),
            pltpu.VMEM_SHARED((NPAD,), jnp.float32),
            pltpu.SemaphoreType.DMA((4,)),
            pltpu.SemaphoreType.DMA((4,)),
            pltpu.SemaphoreType.DMA((2,)),
            pltpu.SemaphoreType.DMA((2,)),
            pltpu.SemaphoreType.DMA((2,)),
        ],
    )
    return f(g_flat, ker2, src, dst, zeros_flat)


def _ker1_body(ea_ref, kw_ref, kb_ref, o_ref):
    o_ref[...] = (jnp.dot(ea_ref[...], kw_ref[0], preferred_element_type=jnp.float32)
                  + kb_ref[0])[None]


def _tc_ker1(edge_attr, kw, kb):
    be = 4000
    return pl.pallas_call(
        _ker1_body,
        grid=(L, E // be),
        in_specs=[
            pl.BlockSpec((be, D_EDGE), lambda l, i: (i, 0)),
            pl.BlockSpec((1, D_EDGE, HID), lambda l, i: (l, 0, 0)),
            pl.BlockSpec((1, 1, HID), lambda l, i: (l, 0, 0)),
        ],
        out_specs=pl.BlockSpec((1, be, HID), lambda l, i: (l, i, 0)),
        out_shape=jax.ShapeDtypeStruct((L, E, HID), jnp.float32),
    )(edge_attr, kw, kb)


def _ker2_body(ea_ref, kw_ref, kb_ref, o_ref):
    o_ref[...] = jnp.dot(ea_ref[...], kw_ref[...], preferred_element_type=jnp.float32) + kb_ref[...]


def _tc_ker2(edge_attr, kw2, kb2):
    be = 8000
    return pl.pallas_call(
        _ker2_body,
        grid=(E // be,),
        in_specs=[
            pl.BlockSpec((be, D_EDGE), lambda i: (i, 0)),
            pl.BlockSpec((D_EDGE, L), lambda i: (0, 0)),
            pl.BlockSpec((1, L), lambda i: (0, 0)),
        ],
        out_specs=pl.BlockSpec((be, L), lambda i: (i, 0)),
        out_shape=jax.ShapeDtypeStruct((E, L), jnp.float32),
    )(edge_attr, kw2, kb2)


def _pe_body(deg_ref, x_ref, w1_ref, b1_ref, w2_ref, b2_ref, lw_ref, lb_ref,
             h_ref, dinv_ref):
    deg = jnp.clip(deg_ref[...], 1.0, None)
    dinv_ref[...] = 1.0 / deg
    pef = jnp.log(1.0 + deg)
    a = jax.nn.relu(pef * w1_ref[...] + b1_ref[...])
    pe = jnp.dot(a, w2_ref[...], preferred_element_type=jnp.float32) + b2_ref[...]
    h0 = x_ref[...] + pe
    h_ref[...] = jnp.dot(h0, lw_ref[...], preferred_element_type=jnp.float32) + lb_ref[...]


def _tc_pe(deg_col, x, w1, b1, w2, b2, lw, lb):
    return pl.pallas_call(
        _pe_body,
        grid=(N // _BN,),
        in_specs=[
            pl.BlockSpec((_BN, 1), lambda i: (i, 0)),
            pl.BlockSpec((_BN, IN_CH), lambda i: (i, 0)),
            pl.BlockSpec((1, HID), lambda i: (0, 0)),
            pl.BlockSpec((1, HID), lambda i: (0, 0)),
            pl.BlockSpec((HID, IN_CH), lambda i: (0, 0)),
            pl.BlockSpec((1, IN_CH), lambda i: (0, 0)),
            pl.BlockSpec((IN_CH, HID), lambda i: (0, 0)),
            pl.BlockSpec((1, HID), lambda i: (0, 0)),
        ],
        out_specs=[
            pl.BlockSpec((_BN, HID), lambda i: (i, 0)),
            pl.BlockSpec((_BN, 1), lambda i: (i, 0)),
        ],
        out_shape=[
            jax.ShapeDtypeStruct((N, HID), jnp.float32),
            jax.ShapeDtypeStruct((N, 1), jnp.float32),
        ],
    )(deg_col, x, w1, b1, w2, b2, lw, lb)


def _upd_body(h_ref, aggp_ref, dinv_ref, sw_ref, sb_ref, o_ref):
    agg = (aggp_ref[0] + aggp_ref[1]) * dinv_ref[...]
    o_ref[...] = jax.nn.relu(
        jnp.dot(h_ref[...], sw_ref[...], preferred_element_type=jnp.float32)
        + sb_ref[...] + agg)


def _tc_update(h, aggp, dinv, sw, sb):
    return pl.pallas_call(
        _upd_body,
        grid=(N // _BN,),
        in_specs=[
            pl.BlockSpec((_BN, HID), lambda i: (i, 0)),
            pl.BlockSpec((2, _BN, HID), lambda i: (0, i, 0)),
            pl.BlockSpec((_BN, 1), lambda i: (i, 0)),
            pl.BlockSpec((HID, HID), lambda i: (0, 0)),
            pl.BlockSpec((1, HID), lambda i: (0, 0)),
        ],
        out_specs=pl.BlockSpec((_BN, HID), lambda i: (i, 0)),
        out_shape=jax.ShapeDtypeStruct((N, HID), jnp.float32),
    )(h, aggp, dinv, sw, sb)


def _updlin_body(h_ref, aggp_ref, dinv_ref, sw_ref, sb_ref, lw_ref, lb_ref, o_ref):
    agg = (aggp_ref[0] + aggp_ref[1]) * dinv_ref[...]
    h1 = jax.nn.relu(
        jnp.dot(h_ref[...], sw_ref[...], preferred_element_type=jnp.float32)
        + sb_ref[...] + agg)
    o_ref[...] = jnp.dot(h1, lw_ref[...], preferred_element_type=jnp.float32) + lb_ref[...]


def _tc_update_lin(h, aggp, dinv, sw, sb, lw, lb):
    return pl.pallas_call(
        _updlin_body,
        grid=(N // _BN,),
        in_specs=[
            pl.BlockSpec((_BN, HID), lambda i: (i, 0)),
            pl.BlockSpec((2, _BN, HID), lambda i: (0, i, 0)),
            pl.BlockSpec((_BN, 1), lambda i: (i, 0)),
            pl.BlockSpec((HID, HID), lambda i: (0, 0)),
            pl.BlockSpec((1, HID), lambda i: (0, 0)),
            pl.BlockSpec((HID, OUT_CH), lambda i: (0, 0)),
            pl.BlockSpec((1, OUT_CH), lambda i: (0, 0)),
        ],
        out_specs=pl.BlockSpec((_BN, OUT_CH), lambda i: (i, 0)),
        out_shape=jax.ShapeDtypeStruct((N, OUT_CH), jnp.float32),
    )(h, aggp, dinv, sw, sb, lw, lb)


def _upd2_body(g_ref, aggp_ref, dinvf_ref, sw_ref, sb_ref, o_ref):
    agg = jnp.sum(aggp_ref[...], axis=0) * dinvf_ref[...]
    o_ref[...] = jax.nn.relu(g_ref[...] * sw_ref[0, 0] + sb_ref[0, 0] + agg)


def _tc_update2(g_flat2d, aggp, dinv_flat2d, sw_s, sb_s):
    return pl.pallas_call(
        _upd2_body,
        grid=(1,),
        in_specs=[
            pl.BlockSpec((NROW, 128), lambda i: (0, 0)),
            pl.BlockSpec((2, NROW, 128), lambda i: (0, 0, 0)),
            pl.BlockSpec((NROW, 128), lambda i: (0, 0)),
            pl.BlockSpec((1, 1), lambda i: (0, 0)),
            pl.BlockSpec((1, 1), lambda i: (0, 0)),
        ],
        out_specs=pl.BlockSpec((NROW, 128), lambda i: (0, 0)),
        out_shape=jax.ShapeDtypeStruct((NROW, 128), jnp.float32),
    )(g_flat2d, aggp, dinv_flat2d, sw_s, sb_s)


def _red2_body(p_ref, o_ref):
    o_ref[...] = jnp.sum(p_ref[...], axis=0)


def _tc_reduce2(p):
    return pl.pallas_call(
        _red2_body,
        grid=(1,),
        in_specs=[pl.BlockSpec((2, NROW, 128), lambda i: (0, 0, 0))],
        out_specs=pl.BlockSpec((NROW, 128), lambda i: (0, 0)),
        out_shape=jax.ShapeDtypeStruct((NROW, 128), jnp.float32),
    )(p)


def kernel(x, edge_index, edge_attr, pe_w1, pe_b1, pe_w2, pe_b2,
           c1_lin_w, c1_lin_b, c1_kw, c1_kb, c1_sw, c1_sb,
           c2_lin_w, c2_lin_b, c2_kw, c2_kb, c2_sw, c2_sb):
    src = edge_index[0]
    dst = edge_index[1]
    zeros_h = jnp.zeros((NR, HID), jnp.float32)
    zeros_flat = jnp.zeros((NPAD,), jnp.float32)

    degp = _sc_deg(dst, zeros_flat)
    deg = _tc_reduce2(degp.reshape(2, NROW, 128))
    deg_col = deg.reshape(NPAD)[:N, None]

    ker1 = _tc_ker1(edge_attr, c1_kw, c1_kb[:, None, :])
    ker2 = _tc_ker2(edge_attr, jnp.transpose(c2_kw, (1, 2, 0)).reshape(D_EDGE, L),
                    c2_kb.reshape(1, L))

    h, dinv = _tc_pe(deg_col, x, pe_w1, pe_b1[None, :], pe_w2, pe_b2[None, :],
                     c1_lin_w, c1_lin_b[None, :])

    for l in range(L):
        aggp = _sc_agg(h, ker1[l], src, dst, zeros_h)
        if l < L - 1:
            h = _tc_update(h, aggp, dinv, c1_sw[l], c1_sb[l][None, :])
        else:
            g = _tc_update_lin(h, aggp, dinv, c1_sw[l], c1_sb[l][None, :],
                               c2_lin_w, c2_lin_b[None, :])

    dinv_flat2d = jnp.pad(dinv[:, 0], (0, NPAD - N)).reshape(NROW, 128)
    g_flat = jnp.pad(g[:, 0], (0, NPAD - N))
    for l in range(L):
        aggp2 = _sc_agg2(g_flat, ker2[:, l], src, dst, zeros_flat)
        g2d = _tc_update2(g_flat.reshape(NROW, 128),
                          aggp2.reshape(2, NROW, 128), dinv_flat2d,
                          c2_sw[l], c2_sb[l].reshape(1, 1))
        g_flat = g2d.reshape(NPAD)

    return g_flat[:N, None]


# speedup vs baseline: 6.2299x; 1.4825x over previous
import functools

import jax
import jax.numpy as jnp
from jax import lax
from jax.experimental import pallas as pl
from jax.experimental.pallas import tpu as pltpu
from jax.experimental.pallas import tpu_sc as plsc

N = 10000
E = 320000
IN_CH = 128
HID = 128
OUT_CH = 1
D_EDGE = 16
L = 2

NPAD = 10240
NROW = NPAD // 128
NW = 32
CH = 128
NCHUNK = E // CH
FULL_ROUNDS = NCHUNK // NW
TAIL = NCHUNK - FULL_ROUNDS * NW
CH1 = 64
NCHUNK1 = E // CH1
ROUNDS1 = NCHUNK1 // NW
TAIL1 = NCHUNK1 - ROUNDS1 * NW
NR = 10112
RPT = NR // 16
EPT = E // NW
FPT = NPAD // 16

_BN = 1000


def _sc_mesh():
    return plsc.VectorSubcoreMesh(core_axis_name="c", subcore_axis_name="s",
                                  num_cores=2, num_subcores=16)


def _deg_body(dst_hbm, zeros_hbm, out_hbm, dstb, ones_v, deg_sh, s_dst, s_s):
    c = lax.axis_index("c")
    s = lax.axis_index("s")
    wid = s * 2 + c
    cnt = jnp.where(wid < TAIL, FULL_ROUNDS + 1, FULL_ROUNDS)

    def ob(i, _):
        ones_v[pl.ds(i * 16, 16)] = jnp.full((16,), 1.0, jnp.float32)
        return 0
    lax.fori_loop(0, CH // 16, ob, 0)

    pltpu.sync_copy(zeros_hbm.at[pl.ds(s * FPT, FPT)], deg_sh.at[pl.ds(s * FPT, FPT)])
    plsc.subcore_barrier()

    def e0_of(r):
        return (wid + r * NW) * CH

    def fire_idx(r):
        b4 = lax.rem(r, 4)
        pltpu.async_copy(dst_hbm.at[pl.ds(e0_of(r), CH)], dstb.at[b4], s_dst.at[b4])

    fire_idx(0)
    @pl.when(cnt > 1)
    def _():
        fire_idx(1)

    def body(g, _):
        b2 = lax.rem(g, 2)
        b4 = lax.rem(g, 4)
        pltpu.make_async_copy(dst_hbm.at[pl.ds(0, CH)], dstb.at[b4], s_dst.at[b4]).wait()
        @pl.when(g >= 2)
        def _():
            pltpu.make_async_copy(ones_v, deg_sh.at[dstb.at[b4]], s_s.at[b2]).wait()
        pltpu.async_copy(ones_v, deg_sh.at[dstb.at[b4]], s_s.at[b2], add=True)
        @pl.when(g + 2 < cnt)
        def _():
            fire_idx(g + 2)
        return 0
    lax.fori_loop(0, cnt, body, 0)

    def drain(g):
        b2 = lax.rem(g, 2)
        b4 = lax.rem(g, 4)
        pltpu.make_async_copy(ones_v, deg_sh.at[dstb.at[b4]], s_s.at[b2]).wait()
    drain(cnt - 2)
    drain(cnt - 1)

    plsc.subcore_barrier()
    pltpu.sync_copy(deg_sh.at[pl.ds(s * FPT, FPT)], out_hbm.at[c, pl.ds(s * FPT, FPT)])


def _sc_deg(dst, zeros_flat):
    f = pl.kernel(
        _deg_body,
        out_type=jax.ShapeDtypeStruct((2, NPAD), jnp.float32),
        mesh=_sc_mesh(),
        scratch_types=[
            pltpu.VMEM((4, CH), jnp.int32),
            pltpu.VMEM((CH,), jnp.float32),
            pltpu.VMEM_SHARED((NPAD,), jnp.float32),
            pltpu.SemaphoreType.DMA((4,)),
            pltpu.SemaphoreType.DMA((2,)),
        ],
    )
    return f(dst, zeros_flat)


def _agg_body(h_hbm, ker_hbm, src_hbm, dst_hbm, zeros_hbm, out_hbm,
              srcb, dstb, hs, kerb, msg, agg_sh,
              s_src, s_dst, s_h, s_k, s_s):
    c = lax.axis_index("c")
    s = lax.axis_index("s")
    wid = s * 2 + c
    cnt = jnp.where(wid < TAIL1, ROUNDS1 + 1, ROUNDS1)

    pltpu.sync_copy(zeros_hbm.at[pl.ds(s * RPT, RPT)], agg_sh.at[pl.ds(s * RPT, RPT)])
    plsc.subcore_barrier()

    def e0_of(r):
        return (wid + r * NW) * CH1

    def fire_idx(r):
        b4 = lax.rem(r, 4)
        pltpu.async_copy(src_hbm.at[pl.ds(e0_of(r), CH1)], srcb.at[b4], s_src.at[b4])
        pltpu.async_copy(dst_hbm.at[pl.ds(e0_of(r), CH1)], dstb.at[b4], s_dst.at[b4])

    def fire_ker(r):
        b2 = lax.rem(r, 2)
        pltpu.async_copy(ker_hbm.at[pl.ds(e0_of(r), CH1)], kerb.at[b2], s_k.at[b2])

    def fire_gather(r):
        b4 = lax.rem(r, 4)
        b2 = lax.rem(r, 2)
        pltpu.make_async_copy(src_hbm.at[pl.ds(0, CH1)], srcb.at[b4], s_src.at[b4]).wait()
        pltpu.async_copy(h_hbm.at[srcb.at[b4]], hs.at[b2], s_h.at[b2])

    fire_idx(0)
    fire_ker(0)
    @pl.when(cnt > 1)
    def _():
        fire_idx(1)
        fire_ker(1)
    fire_gather(0)

    def body(g, _):
        b2 = lax.rem(g, 2)
        b4 = lax.rem(g, 4)

        @pl.when(g + 1 < cnt)
        def _():
            fire_gather(g + 1)

        pltpu.make_async_copy(h_hbm.at[srcb.at[b4]], hs.at[b2], s_h.at[b2]).wait()
        pltpu.make_async_copy(ker_hbm.at[pl.ds(0, CH1)], kerb.at[b2], s_k.at[b2]).wait()
        @pl.when(g >= 2)
        def _():
            pltpu.make_async_copy(msg.at[b2], agg_sh.at[dstb.at[b4]], s_s.at[b2]).wait()
        pltpu.make_async_copy(dst_hbm.at[pl.ds(0, CH1)], dstb.at[b4], s_dst.at[b4]).wait()

        @plsc.parallel_loop(0, CH1, unroll=4)
        def _(i):
            for q in range(HID // 16):
                sl = pl.ds(q * 16, 16)
                msg[b2, i, sl] = hs[b2, i, sl] * kerb[b2, i, sl]

        pltpu.async_copy(msg.at[b2], agg_sh.at[dstb.at[b4]], s_s.at[b2], add=True)

        @pl.when(g + 2 < cnt)
        def _():
            fire_idx(g + 2)
            fire_ker(g + 2)
        return 0
    lax.fori_loop(0, cnt, body, 0)

    def drain(g):
        b2 = lax.rem(g, 2)
        b4 = lax.rem(g, 4)
        pltpu.make_async_copy(msg.at[b2], agg_sh.at[dstb.at[b4]], s_s.at[b2]).wait()
    drain(cnt - 2)
    drain(cnt - 1)

    plsc.subcore_barrier()
    pltpu.sync_copy(agg_sh.at[pl.ds(s * RPT, RPT)], out_hbm.at[c, pl.ds(s * RPT, RPT)])


def _sc_agg(h, ker, src, dst, zeros_h):
    f = pl.kernel(
        _agg_body,
        out_type=jax.ShapeDtypeStruct((2, NR, HID), jnp.float32),
        mesh=_sc_mesh(),
        scratch_types=[
            pltpu.VMEM((4, CH1), jnp.int32),
            pltpu.VMEM((4, CH1), jnp.int32),
            pltpu.VMEM((2, CH1, HID), jnp.float32),
            pltpu.VMEM((2, CH1, HID), jnp.float32),
            pltpu.VMEM((2, CH1, HID), jnp.float32),
            pltpu.VMEM_SHARED((NR, HID), jnp.float32),
            pltpu.SemaphoreType.DMA((4,)),
            pltpu.SemaphoreType.DMA((4,)),
            pltpu.SemaphoreType.DMA((2,)),
            pltpu.SemaphoreType.DMA((2,)),
            pltpu.SemaphoreType.DMA((2,)),
        ],
    )
    return f(h, ker, src, dst, zeros_h)


def _agg2_body(g_hbm, ker_hbm, src_hbm, dst_hbm, zeros_hbm, out_hbm,
               srcb, dstb, hsb, kerb, msgb, agg_sh,
               s_src, s_dst, s_h, s_k, s_s):
    c = lax.axis_index("c")
    s = lax.axis_index("s")
    wid = s * 2 + c
    cnt = jnp.where(wid < TAIL, FULL_ROUNDS + 1, FULL_ROUNDS)

    pltpu.sync_copy(zeros_hbm.at[pl.ds(s * FPT, FPT)], agg_sh.at[pl.ds(s * FPT, FPT)])
    plsc.subcore_barrier()

    def e0_of(r):
        return (wid + r * NW) * CH

    def fire_idx(r):
        b4 = lax.rem(r, 4)
        pltpu.async_copy(src_hbm.at[pl.ds(e0_of(r), CH)], srcb.at[b4], s_src.at[b4])
        pltpu.async_copy(dst_hbm.at[pl.ds(e0_of(r), CH)], dstb.at[b4], s_dst.at[b4])

    def fire_ker(r):
        b2 = lax.rem(r, 2)
        pltpu.async_copy(ker_hbm.at[pl.ds(e0_of(r), CH)], kerb.at[b2], s_k.at[b2])

    def fire_gather(r):
        b4 = lax.rem(r, 4)
        b2 = lax.rem(r, 2)
        pltpu.make_async_copy(src_hbm.at[pl.ds(0, CH)], srcb.at[b4], s_src.at[b4]).wait()
        pltpu.async_copy(g_hbm.at[srcb.at[b4]], hsb.at[b2], s_h.at[b2])

    fire_idx(0)
    fire_ker(0)
    @pl.when(cnt > 1)
    def _():
        fire_idx(1)
        fire_ker(1)
    fire_gather(0)

    def body(g, _):
        b2 = lax.rem(g, 2)
        b4 = lax.rem(g, 4)

        @pl.when(g + 1 < cnt)
        def _():
            fire_gather(g + 1)

        pltpu.make_async_copy(g_hbm.at[srcb.at[b4]], hsb.at[b2], s_h.at[b2]).wait()
        pltpu.make_async_copy(ker_hbm.at[pl.ds(0, CH)], kerb.at[b2], s_k.at[b2]).wait()
        @pl.when(g >= 2)
        def _():
            pltpu.make_async_copy(msgb.at[b2], agg_sh.at[dstb.at[b4]], s_s.at[b2]).wait()
        pltpu.make_async_copy(dst_hbm.at[pl.ds(0, CH)], dstb.at[b4], s_dst.at[b4]).wait()

        @plsc.parallel_loop(0, CH // 16, unroll=4)
        def _(i):
            sl = pl.ds(i * 16, 16)
            msgb[b2, sl] = hsb[b2, sl] * kerb[b2, sl]

        pltpu.async_copy(msgb.at[b2], agg_sh.at[dstb.at[b4]], s_s.at[b2], add=True)

        @pl.when(g + 2 < cnt)
        def _():
            fire_idx(g + 2)
            fire_ker(g + 2)
        return 0
    lax.fori_loop(0, cnt, body, 0)

    def drain(g):
        b2 = lax.rem(g, 2)
        b4 = lax.rem(g, 4)
        pltpu.make_async_copy(msgb.at[b2], agg_sh.at[dstb.at[b4]], s_s.at[b2]).wait()
    drain(cnt - 2)
    drain(cnt - 1)

    plsc.subcore_barrier()
    pltpu.sync_copy(agg_sh.at[pl.ds(s * FPT, FPT)], out_hbm.at[c, pl.ds(s * FPT, FPT)])


def _sc_agg2(g_flat, ker2, src, dst, zeros_flat):
    f = pl.kernel(
        _agg2_body,
        out_type=jax.ShapeDtypeStruct((2, NPAD), jnp.float32),
        mesh=_sc_mesh(),
        scratch_types=[
            pltpu.VMEM((4, CH), jnp.int32),
            pltpu.VMEM((4, CH), jnp.int32),
            pltpu.VMEM((2, CH), jnp.float32),
            pltpu.VMEM((2, CH), jnp.float32),
            pltpu.VMEM((2, CH), jnp.float32),
            pltpu.VMEM_SHARED((NPAD,), jnp.float32),
            pltpu.SemaphoreType.DMA((4,)),
            pltpu.SemaphoreType.DMA((4,)),
            pltpu.SemaphoreType.DMA((2,)),
            pltpu.SemaphoreType.DMA((2,)),
            pltpu.SemaphoreType.DMA((2,)),
        ],
    )
    return f(g_flat, ker2, src, dst, zeros_flat)


def _ker1_body(ea_ref, kw_ref, kb_ref, o_ref):
    o_ref[...] = (jnp.dot(ea_ref[...], kw_ref[0], preferred_element_type=jnp.float32)
                  + kb_ref[0])[None]


def _tc_ker1(edge_attr, kw, kb):
    be = 4000
    return pl.pallas_call(
        _ker1_body,
        grid=(L, E // be),
        in_specs=[
            pl.BlockSpec((be, D_EDGE), lambda l, i: (i, 0)),
            pl.BlockSpec((1, D_EDGE, HID), lambda l, i: (l, 0, 0)),
            pl.BlockSpec((1, 1, HID), lambda l, i: (l, 0, 0)),
        ],
        out_specs=pl.BlockSpec((1, be, HID), lambda l, i: (l, i, 0)),
        out_shape=jax.ShapeDtypeStruct((L, E, HID), jnp.float32),
    )(edge_attr, kw, kb)


def _ker2_body(ea_ref, kw_ref, kb_ref, o_ref):
    o_ref[...] = jnp.dot(ea_ref[...], kw_ref[...], preferred_element_type=jnp.float32) + kb_ref[...]


def _tc_ker2(edge_attr, kw2, kb2):
    be = 8000
    return pl.pallas_call(
        _ker2_body,
        grid=(E // be,),
        in_specs=[
            pl.BlockSpec((be, D_EDGE), lambda i: (i, 0)),
            pl.BlockSpec((D_EDGE, L), lambda i: (0, 0)),
            pl.BlockSpec((1, L), lambda i: (0, 0)),
        ],
        out_specs=pl.BlockSpec((be, L), lambda i: (i, 0)),
        out_shape=jax.ShapeDtypeStruct((E, L), jnp.float32),
    )(edge_attr, kw2, kb2)


def _pe_body(deg_ref, x_ref, w1_ref, b1_ref, w2_ref, b2_ref, lw_ref, lb_ref,
             h_ref, dinv_ref):
    deg = jnp.clip(deg_ref[...], 1.0, None)
    dinv_ref[...] = 1.0 / deg
    pef = jnp.log(1.0 + deg)
    a = jax.nn.relu(pef * w1_ref[...] + b1_ref[...])
    pe = jnp.dot(a, w2_ref[...], preferred_element_type=jnp.float32) + b2_ref[...]
    h0 = x_ref[...] + pe
    h_ref[...] = jnp.dot(h0, lw_ref[...], preferred_element_type=jnp.float32) + lb_ref[...]


def _tc_pe(deg_col, x, w1, b1, w2, b2, lw, lb):
    return pl.pallas_call(
        _pe_body,
        grid=(N // _BN,),
        in_specs=[
            pl.BlockSpec((_BN, 1), lambda i: (i, 0)),
            pl.BlockSpec((_BN, IN_CH), lambda i: (i, 0)),
            pl.BlockSpec((1, HID), lambda i: (0, 0)),
            pl.BlockSpec((1, HID), lambda i: (0, 0)),
            pl.BlockSpec((HID, IN_CH), lambda i: (0, 0)),
            pl.BlockSpec((1, IN_CH), lambda i: (0, 0)),
            pl.BlockSpec((IN_CH, HID), lambda i: (0, 0)),
            pl.BlockSpec((1, HID), lambda i: (0, 0)),
        ],
        out_specs=[
            pl.BlockSpec((_BN, HID), lambda i: (i, 0)),
            pl.BlockSpec((_BN, 1), lambda i: (i, 0)),
        ],
        out_shape=[
            jax.ShapeDtypeStruct((N, HID), jnp.float32),
            jax.ShapeDtypeStruct((N, 1), jnp.float32),
        ],
    )(deg_col, x, w1, b1, w2, b2, lw, lb)


def _upd_body(h_ref, aggp_ref, dinv_ref, sw_ref, sb_ref, o_ref):
    agg = (aggp_ref[0] + aggp_ref[1]) * dinv_ref[...]
    o_ref[...] = jax.nn.relu(
        jnp.dot(h_ref[...], sw_ref[...], preferred_element_type=jnp.float32)
        + sb_ref[...] + agg)


def _tc_update(h, aggp, dinv, sw, sb):
    return pl.pallas_call(
        _upd_body,
        grid=(N // _BN,),
        in_specs=[
            pl.BlockSpec((_BN, HID), lambda i: (i, 0)),
            pl.BlockSpec((2, _BN, HID), lambda i: (0, i, 0)),
            pl.BlockSpec((_BN, 1), lambda i: (i, 0)),
            pl.BlockSpec((HID, HID), lambda i: (0, 0)),
            pl.BlockSpec((1, HID), lambda i: (0, 0)),
        ],
        out_specs=pl.BlockSpec((_BN, HID), lambda i: (i, 0)),
        out_shape=jax.ShapeDtypeStruct((N, HID), jnp.float32),
    )(h, aggp, dinv, sw, sb)


def _updlin_body(h_ref, aggp_ref, dinv_ref, sw_ref, sb_ref, lw_ref, lb_ref, o_ref):
    agg = (aggp_ref[0] + aggp_ref[1]) * dinv_ref[...]
    h1 = jax.nn.relu(
        jnp.dot(h_ref[...], sw_ref[...], preferred_element_type=jnp.float32)
        + sb_ref[...] + agg)
    o_ref[...] = jnp.dot(h1, lw_ref[...], preferred_element_type=jnp.float32) + lb_ref[...]


def _tc_update_lin(h, aggp, dinv, sw, sb, lw, lb):
    return pl.pallas_call(
        _updlin_body,
        grid=(N // _BN,),
        in_specs=[
            pl.BlockSpec((_BN, HID), lambda i: (i, 0)),
            pl.BlockSpec((2, _BN, HID), lambda i: (0, i, 0)),
            pl.BlockSpec((_BN, 1), lambda i: (i, 0)),
            pl.BlockSpec((HID, HID), lambda i: (0, 0)),
            pl.BlockSpec((1, HID), lambda i: (0, 0)),
            pl.BlockSpec((HID, OUT_CH), lambda i: (0, 0)),
            pl.BlockSpec((1, OUT_CH), lambda i: (0, 0)),
        ],
        out_specs=pl.BlockSpec((_BN, OUT_CH), lambda i: (i, 0)),
        out_shape=jax.ShapeDtypeStruct((N, OUT_CH), jnp.float32),
    )(h, aggp, dinv, sw, sb, lw, lb)


def _upd2_body(g_ref, aggp_ref, dinvf_ref, sw_ref, sb_ref, o_ref):
    agg = jnp.sum(aggp_ref[...], axis=0) * dinvf_ref[...]
    o_ref[...] = jax.nn.relu(g_ref[...] * sw_ref[0, 0] + sb_ref[0, 0] + agg)


def _tc_update2(g_flat2d, aggp, dinv_flat2d, sw_s, sb_s):
    return pl.pallas_call(
        _upd2_body,
        grid=(1,),
        in_specs=[
            pl.BlockSpec((NROW, 128), lambda i: (0, 0)),
            pl.BlockSpec((2, NROW, 128), lambda i: (0, 0, 0)),
            pl.BlockSpec((NROW, 128), lambda i: (0, 0)),
            pl.BlockSpec((1, 1), lambda i: (0, 0)),
            pl.BlockSpec((1, 1), lambda i: (0, 0)),
        ],
        out_specs=pl.BlockSpec((NROW, 128), lambda i: (0, 0)),
        out_shape=jax.ShapeDtypeStruct((NROW, 128), jnp.float32),
    )(g_flat2d, aggp, dinv_flat2d, sw_s, sb_s)


def _red2_body(p_ref, o_ref):
    o_ref[...] = jnp.sum(p_ref[...], axis=0)


def _tc_reduce2(p):
    return pl.pallas_call(
        _red2_body,
        grid=(1,),
        in_specs=[pl.BlockSpec((2, NROW, 128), lambda i: (0, 0, 0))],
        out_specs=pl.BlockSpec((NROW, 128), lambda i: (0, 0)),
        out_shape=jax.ShapeDtypeStruct((NROW, 128), jnp.float32),
    )(p)


def kernel(x, edge_index, edge_attr, pe_w1, pe_b1, pe_w2, pe_b2,
           c1_lin_w, c1_lin_b, c1_kw, c1_kb, c1_sw, c1_sb,
           c2_lin_w, c2_lin_b, c2_kw, c2_kb, c2_sw, c2_sb):
    src = edge_index[0]
    dst = edge_index[1]
    zeros_h = jnp.zeros((NR, HID), jnp.float32)
    zeros_flat = jnp.zeros((NPAD,), jnp.float32)

    degp = _sc_deg(dst, zeros_flat)
    deg = _tc_reduce2(degp.reshape(2, NROW, 128))
    deg_col = deg.reshape(NPAD)[:N, None]

    ker1 = _tc_ker1(edge_attr, c1_kw, c1_kb[:, None, :])
    ker2 = _tc_ker2(edge_attr, jnp.transpose(c2_kw, (1, 2, 0)).reshape(D_EDGE, L),
                    c2_kb.reshape(1, L))

    h, dinv = _tc_pe(deg_col, x, pe_w1, pe_b1[None, :], pe_w2, pe_b2[None, :],
                     c1_lin_w, c1_lin_b[None, :])

    for l in range(L):
        aggp = _sc_agg(h, ker1[l], src, dst, zeros_h)
        if l < L - 1:
            h = _tc_update(h, aggp, dinv, c1_sw[l], c1_sb[l][None, :])
        else:
            g = _tc_update_lin(h, aggp, dinv, c1_sw[l], c1_sb[l][None, :],
                               c2_lin_w, c2_lin_b[None, :])

    dinv_flat2d = jnp.pad(dinv[:, 0], (0, NPAD - N)).reshape(NROW, 128)
    g_flat = jnp.pad(g[:, 0], (0, NPAD - N))
    for l in range(L):
        aggp2 = _sc_agg2(g_flat, ker2[:, l], src, dst, zeros_flat)
        g2d = _tc_update2(g_flat.reshape(NROW, 128),
                          aggp2.reshape(2, NROW, 128), dinv_flat2d,
                          c2_sw[l], c2_sb[l].reshape(1, 1))
        g_flat = g2d.reshape(NPAD)

    return g_flat[:N, None]


# speedup vs baseline: 7.3962x; 1.1872x over previous
import functools

import jax
import jax.numpy as jnp
from jax import lax
from jax.experimental import pallas as pl
from jax.experimental.pallas import tpu as pltpu
from jax.experimental.pallas import tpu_sc as plsc

N = 10000
E = 320000
IN_CH = 128
HID = 128
OUT_CH = 1
D_EDGE = 16
L = 2

NPAD = 10240
NROW = NPAD // 128
NW = 32
CH = 128
NCHUNK = E // CH
FULL_ROUNDS = NCHUNK // NW
TAIL = NCHUNK - FULL_ROUNDS * NW
CH1 = 64
NCHUNK1 = E // CH1
ROUNDS1 = NCHUNK1 // NW
TAIL1 = NCHUNK1 - ROUNDS1 * NW
NR = 10112
RPT = NR // 16
EPT = E // NW
FPT = NPAD // 16

_BN = 1000


def _sc_mesh():
    return plsc.VectorSubcoreMesh(core_axis_name="c", subcore_axis_name="s",
                                  num_cores=2, num_subcores=16)


def _deg_body(dst_hbm, zeros_hbm, out_hbm, dstb, ones_v, deg_sh, s_dst, s_s):
    c = lax.axis_index("c")
    s = lax.axis_index("s")
    wid = s * 2 + c
    cnt = jnp.where(wid < TAIL, FULL_ROUNDS + 1, FULL_ROUNDS)

    def ob(i, _):
        ones_v[pl.ds(i * 16, 16)] = jnp.full((16,), 1.0, jnp.float32)
        return 0
    lax.fori_loop(0, CH // 16, ob, 0)

    pltpu.sync_copy(zeros_hbm.at[pl.ds(s * FPT, FPT)], deg_sh.at[pl.ds(s * FPT, FPT)])
    plsc.subcore_barrier()

    def e0_of(r):
        return (wid + r * NW) * CH

    def fire_idx(r):
        b4 = lax.rem(r, 4)
        pltpu.async_copy(dst_hbm.at[pl.ds(e0_of(r), CH)], dstb.at[b4], s_dst.at[b4])

    fire_idx(0)
    @pl.when(cnt > 1)
    def _():
        fire_idx(1)

    def body(g, _):
        b2 = lax.rem(g, 2)
        b4 = lax.rem(g, 4)
        pltpu.make_async_copy(dst_hbm.at[pl.ds(0, CH)], dstb.at[b4], s_dst.at[b4]).wait()
        @pl.when(g >= 2)
        def _():
            pltpu.make_async_copy(ones_v, deg_sh.at[dstb.at[b4]], s_s.at[b2]).wait()
        pltpu.async_copy(ones_v, deg_sh.at[dstb.at[b4]], s_s.at[b2], add=True)
        @pl.when(g + 2 < cnt)
        def _():
            fire_idx(g + 2)
        return 0
    lax.fori_loop(0, cnt, body, 0)

    def drain(g):
        b2 = lax.rem(g, 2)
        b4 = lax.rem(g, 4)
        pltpu.make_async_copy(ones_v, deg_sh.at[dstb.at[b4]], s_s.at[b2]).wait()
    drain(cnt - 2)
    drain(cnt - 1)

    plsc.subcore_barrier()
    pltpu.sync_copy(deg_sh.at[pl.ds(s * FPT, FPT)], out_hbm.at[c, pl.ds(s * FPT, FPT)])


def _sc_deg(dst, zeros_flat):
    f = pl.kernel(
        _deg_body,
        out_type=jax.ShapeDtypeStruct((2, NPAD), jnp.float32),
        mesh=_sc_mesh(),
        scratch_types=[
            pltpu.VMEM((4, CH), jnp.int32),
            pltpu.VMEM((CH,), jnp.float32),
            pltpu.VMEM_SHARED((NPAD,), jnp.float32),
            pltpu.SemaphoreType.DMA((4,)),
            pltpu.SemaphoreType.DMA((2,)),
        ],
    )
    return f(dst, zeros_flat)


def _agg_body(h_hbm, ker_hbm, src_hbm, dst_hbm, zeros_hbm, out_hbm,
              srcb, dstb, hs, kerb, msg, agg_sh,
              s_src, s_dst, s_h, s_k, s_s):
    c = lax.axis_index("c")
    s = lax.axis_index("s")
    wid = s * 2 + c
    cnt = jnp.where(wid < TAIL1, ROUNDS1 + 1, ROUNDS1)

    pltpu.sync_copy(zeros_hbm.at[pl.ds(s * RPT, RPT)], agg_sh.at[pl.ds(s * RPT, RPT)])
    plsc.subcore_barrier()

    def e0_of(r):
        return (wid + r * NW) * CH1

    def fire_idx(r):
        b4 = lax.rem(r, 4)
        pltpu.async_copy(src_hbm.at[pl.ds(e0_of(r), CH1)], srcb.at[b4], s_src.at[b4])
        pltpu.async_copy(dst_hbm.at[pl.ds(e0_of(r), CH1)], dstb.at[b4], s_dst.at[b4])

    def fire_ker(r):
        b2 = lax.rem(r, 2)
        pltpu.async_copy(ker_hbm.at[pl.ds(e0_of(r), CH1)], kerb.at[b2], s_k.at[b2])

    def fire_gather(r):
        b4 = lax.rem(r, 4)
        b2 = lax.rem(r, 2)
        pltpu.make_async_copy(src_hbm.at[pl.ds(0, CH1)], srcb.at[b4], s_src.at[b4]).wait()
        pltpu.async_copy(h_hbm.at[srcb.at[b4]], hs.at[b2], s_h.at[b2])

    fire_idx(0)
    fire_ker(0)
    @pl.when(cnt > 1)
    def _():
        fire_idx(1)
        fire_ker(1)
    fire_gather(0)

    def body(g, _):
        b2 = lax.rem(g, 2)
        b4 = lax.rem(g, 4)

        @pl.when(g + 1 < cnt)
        def _():
            fire_gather(g + 1)

        pltpu.make_async_copy(h_hbm.at[srcb.at[b4]], hs.at[b2], s_h.at[b2]).wait()
        pltpu.make_async_copy(ker_hbm.at[pl.ds(0, CH1)], kerb.at[b2], s_k.at[b2]).wait()
        @pl.when(g >= 2)
        def _():
            pltpu.make_async_copy(msg.at[b2], agg_sh.at[dstb.at[b4]], s_s.at[b2]).wait()
        pltpu.make_async_copy(dst_hbm.at[pl.ds(0, CH1)], dstb.at[b4], s_dst.at[b4]).wait()

        @plsc.parallel_loop(0, CH1, unroll=4)
        def _(i):
            for q in range(HID // 32):
                w = kerb[b2, i, pl.ds(q * 16, 16)]
                ka = lax.bitcast_convert_type(lax.shift_left(w, 16), jnp.float32)
                kb_ = lax.bitcast_convert_type(w & jnp.int32(-65536), jnp.float32)
                sa = pl.ds(q * 32, 16)
                sb_ = pl.ds(q * 32 + 16, 16)
                msg[b2, i, sa] = hs[b2, i, sa] * ka
                msg[b2, i, sb_] = hs[b2, i, sb_] * kb_

        pltpu.async_copy(msg.at[b2], agg_sh.at[dstb.at[b4]], s_s.at[b2], add=True)

        @pl.when(g + 2 < cnt)
        def _():
            fire_idx(g + 2)
            fire_ker(g + 2)
        return 0
    lax.fori_loop(0, cnt, body, 0)

    def drain(g):
        b2 = lax.rem(g, 2)
        b4 = lax.rem(g, 4)
        pltpu.make_async_copy(msg.at[b2], agg_sh.at[dstb.at[b4]], s_s.at[b2]).wait()
    drain(cnt - 2)
    drain(cnt - 1)

    plsc.subcore_barrier()
    pltpu.sync_copy(agg_sh.at[pl.ds(s * RPT, RPT)], out_hbm.at[c, pl.ds(s * RPT, RPT)])


def _sc_agg(h, ker, src, dst, zeros_h):
    f = pl.kernel(
        _agg_body,
        out_type=jax.ShapeDtypeStruct((2, NR, HID), jnp.float32),
        mesh=_sc_mesh(),
        scratch_types=[
            pltpu.VMEM((4, CH1), jnp.int32),
            pltpu.VMEM((4, CH1), jnp.int32),
            pltpu.VMEM((2, CH1, HID), jnp.float32),
            pltpu.VMEM((2, CH1, HID // 2), jnp.int32),
            pltpu.VMEM((2, CH1, HID), jnp.float32),
            pltpu.VMEM_SHARED((NR, HID), jnp.float32),
            pltpu.SemaphoreType.DMA((4,)),
            pltpu.SemaphoreType.DMA((4,)),
            pltpu.SemaphoreType.DMA((2,)),
            pltpu.SemaphoreType.DMA((2,)),
            pltpu.SemaphoreType.DMA((2,)),
        ],
    )
    return f(h, ker, src, dst, zeros_h)


def _agg2_body(g_hbm, ker_hbm, src_hbm, dst_hbm, zeros_hbm, out_hbm,
               srcb, dstb, hsb, kerb, msgb, agg_sh,
               s_src, s_dst, s_h, s_k, s_s):
    c = lax.axis_index("c")
    s = lax.axis_index("s")
    wid = s * 2 + c
    cnt = jnp.where(wid < TAIL, FULL_ROUNDS + 1, FULL_ROUNDS)

    pltpu.sync_copy(zeros_hbm.at[pl.ds(s * FPT, FPT)], agg_sh.at[pl.ds(s * FPT, FPT)])
    plsc.subcore_barrier()

    def e0_of(r):
        return (wid + r * NW) * CH

    def fire_idx(r):
        b4 = lax.rem(r, 4)
        pltpu.async_copy(src_hbm.at[pl.ds(e0_of(r), CH)], srcb.at[b4], s_src.at[b4])
        pltpu.async_copy(dst_hbm.at[pl.ds(e0_of(r), CH)], dstb.at[b4], s_dst.at[b4])

    def fire_ker(r):
        b2 = lax.rem(r, 2)
        pltpu.async_copy(ker_hbm.at[pl.ds(e0_of(r), CH)], kerb.at[b2], s_k.at[b2])

    def fire_gather(r):
        b4 = lax.rem(r, 4)
        b2 = lax.rem(r, 2)
        pltpu.make_async_copy(src_hbm.at[pl.ds(0, CH)], srcb.at[b4], s_src.at[b4]).wait()
        pltpu.async_copy(g_hbm.at[srcb.at[b4]], hsb.at[b2], s_h.at[b2])

    fire_idx(0)
    fire_ker(0)
    @pl.when(cnt > 1)
    def _():
        fire_idx(1)
        fire_ker(1)
    fire_gather(0)

    def body(g, _):
        b2 = lax.rem(g, 2)
        b4 = lax.rem(g, 4)

        @pl.when(g + 1 < cnt)
        def _():
            fire_gather(g + 1)

        pltpu.make_async_copy(g_hbm.at[srcb.at[b4]], hsb.at[b2], s_h.at[b2]).wait()
        pltpu.make_async_copy(ker_hbm.at[pl.ds(0, CH)], kerb.at[b2], s_k.at[b2]).wait()
        @pl.when(g >= 2)
        def _():
            pltpu.make_async_copy(msgb.at[b2], agg_sh.at[dstb.at[b4]], s_s.at[b2]).wait()
        pltpu.make_async_copy(dst_hbm.at[pl.ds(0, CH)], dstb.at[b4], s_dst.at[b4]).wait()

        @plsc.parallel_loop(0, CH // 16, unroll=4)
        def _(i):
            sl = pl.ds(i * 16, 16)
            msgb[b2, sl] = hsb[b2, sl] * kerb[b2, sl]

        pltpu.async_copy(msgb.at[b2], agg_sh.at[dstb.at[b4]], s_s.at[b2], add=True)

        @pl.when(g + 2 < cnt)
        def _():
            fire_idx(g + 2)
            fire_ker(g + 2)
        return 0
    lax.fori_loop(0, cnt, body, 0)

    def drain(g):
        b2 = lax.rem(g, 2)
        b4 = lax.rem(g, 4)
        pltpu.make_async_copy(msgb.at[b2], agg_sh.at[dstb.at[b4]], s_s.at[b2]).wait()
    drain(cnt - 2)
    drain(cnt - 1)

    plsc.subcore_barrier()
    pltpu.sync_copy(agg_sh.at[pl.ds(s * FPT, FPT)], out_hbm.at[c, pl.ds(s * FPT, FPT)])


def _sc_agg2(g_flat, ker2, src, dst, zeros_flat):
    f = pl.kernel(
        _agg2_body,
        out_type=jax.ShapeDtypeStruct((2, NPAD), jnp.float32),
        mesh=_sc_mesh(),
        scratch_types=[
            pltpu.VMEM((4, CH), jnp.int32),
            pltpu.VMEM((4, CH), jnp.int32),
            pltpu.VMEM((2, CH), jnp.float32),
            pltpu.VMEM((2, CH), jnp.float32),
            pltpu.VMEM((2, CH), jnp.float32),
            pltpu.VMEM_SHARED((NPAD,), jnp.float32),
            pltpu.SemaphoreType.DMA((4,)),
            pltpu.SemaphoreType.DMA((4,)),
            pltpu.SemaphoreType.DMA((2,)),
            pltpu.SemaphoreType.DMA((2,)),
            pltpu.SemaphoreType.DMA((2,)),
        ],
    )
    return f(g_flat, ker2, src, dst, zeros_flat)


def _ker1_body(ea_ref, kw_ref, kb_ref, o_ref):
    kerf = (jnp.dot(ea_ref[...], kw_ref[...], preferred_element_type=jnp.float32)
            + kb_ref[...])
    words = []
    for q in range(HID // 32):
        a = lax.bitcast_convert_type(kerf[:, q * 32:q * 32 + 16], jnp.int32)
        b = lax.bitcast_convert_type(kerf[:, q * 32 + 16:q * 32 + 32], jnp.int32)
        wa = lax.shift_right_logical(a + 0x8000, 16)
        wb = (b + 0x8000) & jnp.int32(-65536)
        words.append(wa | wb)
    o_ref[...] = jnp.concatenate(words, axis=1)


def _tc_ker1(edge_attr, kw_l, kb_l):
    be = 4000
    return pl.pallas_call(
        _ker1_body,
        grid=(E // be,),
        in_specs=[
            pl.BlockSpec((be, D_EDGE), lambda i: (i, 0)),
            pl.BlockSpec((D_EDGE, HID), lambda i: (0, 0)),
            pl.BlockSpec((1, HID), lambda i: (0, 0)),
        ],
        out_specs=pl.BlockSpec((be, HID // 2), lambda i: (i, 0)),
        out_shape=jax.ShapeDtypeStruct((E, HID // 2), jnp.int32),
    )(edge_attr, kw_l, kb_l)


def _ker2_body(ea_ref, kw_ref, kb_ref, o_ref):
    o_ref[...] = jnp.dot(ea_ref[...], kw_ref[...], preferred_element_type=jnp.float32) + kb_ref[...]


def _tc_ker2(edge_attr, kw2, kb2):
    be = 8000
    return pl.pallas_call(
        _ker2_body,
        grid=(E // be,),
        in_specs=[
            pl.BlockSpec((be, D_EDGE), lambda i: (i, 0)),
            pl.BlockSpec((D_EDGE, L), lambda i: (0, 0)),
            pl.BlockSpec((1, L), lambda i: (0, 0)),
        ],
        out_specs=pl.BlockSpec((be, L), lambda i: (i, 0)),
        out_shape=jax.ShapeDtypeStruct((E, L), jnp.float32),
    )(edge_attr, kw2, kb2)


def _pe_body(deg_ref, x_ref, w1_ref, b1_ref, w2_ref, b2_ref, lw_ref, lb_ref,
             h_ref, dinv_ref):
    deg = jnp.clip(deg_ref[...], 1.0, None)
    dinv_ref[...] = 1.0 / deg
    pef = jnp.log(1.0 + deg)
    a = jax.nn.relu(pef * w1_ref[...] + b1_ref[...])
    pe = jnp.dot(a, w2_ref[...], preferred_element_type=jnp.float32) + b2_ref[...]
    h0 = x_ref[...] + pe
    h_ref[...] = jnp.dot(h0, lw_ref[...], preferred_element_type=jnp.float32) + lb_ref[...]


def _tc_pe(deg_col, x, w1, b1, w2, b2, lw, lb):
    return pl.pallas_call(
        _pe_body,
        grid=(N // _BN,),
        in_specs=[
            pl.BlockSpec((_BN, 1), lambda i: (i, 0)),
            pl.BlockSpec((_BN, IN_CH), lambda i: (i, 0)),
            pl.BlockSpec((1, HID), lambda i: (0, 0)),
            pl.BlockSpec((1, HID), lambda i: (0, 0)),
            pl.BlockSpec((HID, IN_CH), lambda i: (0, 0)),
            pl.BlockSpec((1, IN_CH), lambda i: (0, 0)),
            pl.BlockSpec((IN_CH, HID), lambda i: (0, 0)),
            pl.BlockSpec((1, HID), lambda i: (0, 0)),
        ],
        out_specs=[
            pl.BlockSpec((_BN, HID), lambda i: (i, 0)),
            pl.BlockSpec((_BN, 1), lambda i: (i, 0)),
        ],
        out_shape=[
            jax.ShapeDtypeStruct((N, HID), jnp.float32),
            jax.ShapeDtypeStruct((N, 1), jnp.float32),
        ],
    )(deg_col, x, w1, b1, w2, b2, lw, lb)


def _upd_body(h_ref, aggp_ref, dinv_ref, sw_ref, sb_ref, o_ref):
    agg = (aggp_ref[0] + aggp_ref[1]) * dinv_ref[...]
    o_ref[...] = jax.nn.relu(
        jnp.dot(h_ref[...], sw_ref[...], preferred_element_type=jnp.float32)
        + sb_ref[...] + agg)


def _tc_update(h, aggp, dinv, sw, sb):
    return pl.pallas_call(
        _upd_body,
        grid=(N // _BN,),
        in_specs=[
            pl.BlockSpec((_BN, HID), lambda i: (i, 0)),
            pl.BlockSpec((2, _BN, HID), lambda i: (0, i, 0)),
            pl.BlockSpec((_BN, 1), lambda i: (i, 0)),
            pl.BlockSpec((HID, HID), lambda i: (0, 0)),
            pl.BlockSpec((1, HID), lambda i: (0, 0)),
        ],
        out_specs=pl.BlockSpec((_BN, HID), lambda i: (i, 0)),
        out_shape=jax.ShapeDtypeStruct((N, HID), jnp.float32),
    )(h, aggp, dinv, sw, sb)


def _updlin_body(h_ref, aggp_ref, dinv_ref, sw_ref, sb_ref, lw_ref, lb_ref, o_ref):
    agg = (aggp_ref[0] + aggp_ref[1]) * dinv_ref[...]
    h1 = jax.nn.relu(
        jnp.dot(h_ref[...], sw_ref[...], preferred_element_type=jnp.float32)
        + sb_ref[...] + agg)
    o_ref[...] = jnp.dot(h1, lw_ref[...], preferred_element_type=jnp.float32) + lb_ref[...]


def _tc_update_lin(h, aggp, dinv, sw, sb, lw, lb):
    return pl.pallas_call(
        _updlin_body,
        grid=(N // _BN,),
        in_specs=[
            pl.BlockSpec((_BN, HID), lambda i: (i, 0)),
            pl.BlockSpec((2, _BN, HID), lambda i: (0, i, 0)),
            pl.BlockSpec((_BN, 1), lambda i: (i, 0)),
            pl.BlockSpec((HID, HID), lambda i: (0, 0)),
            pl.BlockSpec((1, HID), lambda i: (0, 0)),
            pl.BlockSpec((HID, OUT_CH), lambda i: (0, 0)),
            pl.BlockSpec((1, OUT_CH), lambda i: (0, 0)),
        ],
        out_specs=pl.BlockSpec((_BN, OUT_CH), lambda i: (i, 0)),
        out_shape=jax.ShapeDtypeStruct((N, OUT_CH), jnp.float32),
    )(h, aggp, dinv, sw, sb, lw, lb)


def _upd2_body(g_ref, aggp_ref, dinvf_ref, sw_ref, sb_ref, o_ref):
    agg = jnp.sum(aggp_ref[...], axis=0) * dinvf_ref[...]
    o_ref[...] = jax.nn.relu(g_ref[...] * sw_ref[0, 0] + sb_ref[0, 0] + agg)


def _tc_update2(g_flat2d, aggp, dinv_flat2d, sw_s, sb_s):
    return pl.pallas_call(
        _upd2_body,
        grid=(1,),
        in_specs=[
            pl.BlockSpec((NROW, 128), lambda i: (0, 0)),
            pl.BlockSpec((2, NROW, 128), lambda i: (0, 0, 0)),
            pl.BlockSpec((NROW, 128), lambda i: (0, 0)),
            pl.BlockSpec((1, 1), lambda i: (0, 0)),
            pl.BlockSpec((1, 1), lambda i: (0, 0)),
        ],
        out_specs=pl.BlockSpec((NROW, 128), lambda i: (0, 0)),
        out_shape=jax.ShapeDtypeStruct((NROW, 128), jnp.float32),
    )(g_flat2d, aggp, dinv_flat2d, sw_s, sb_s)


def _red2_body(p_ref, o_ref):
    o_ref[...] = jnp.sum(p_ref[...], axis=0)


def _tc_reduce2(p):
    return pl.pallas_call(
        _red2_body,
        grid=(1,),
        in_specs=[pl.BlockSpec((2, NROW, 128), lambda i: (0, 0, 0))],
        out_specs=pl.BlockSpec((NROW, 128), lambda i: (0, 0)),
        out_shape=jax.ShapeDtypeStruct((NROW, 128), jnp.float32),
    )(p)


def kernel(x, edge_index, edge_attr, pe_w1, pe_b1, pe_w2, pe_b2,
           c1_lin_w, c1_lin_b, c1_kw, c1_kb, c1_sw, c1_sb,
           c2_lin_w, c2_lin_b, c2_kw, c2_kb, c2_sw, c2_sb):
    src = edge_index[0]
    dst = edge_index[1]
    zeros_h = jnp.zeros((NR, HID), jnp.float32)
    zeros_flat = jnp.zeros((NPAD,), jnp.float32)

    degp = _sc_deg(dst, zeros_flat)
    deg = _tc_reduce2(degp.reshape(2, NROW, 128))
    deg_col = deg.reshape(NPAD)[:N, None]

    ker1 = [_tc_ker1(edge_attr, c1_kw[l], c1_kb[l][None, :]) for l in range(L)]
    ker2 = _tc_ker2(edge_attr, jnp.transpose(c2_kw, (1, 2, 0)).reshape(D_EDGE, L),
                    c2_kb.reshape(1, L))

    h, dinv = _tc_pe(deg_col, x, pe_w1, pe_b1[None, :], pe_w2, pe_b2[None, :],
                     c1_lin_w, c1_lin_b[None, :])

    for l in range(L):
        aggp = _sc_agg(h, ker1[l], src, dst, zeros_h)
        if l < L - 1:
            h = _tc_update(h, aggp, dinv, c1_sw[l], c1_sb[l][None, :])
        else:
            g = _tc_update_lin(h, aggp, dinv, c1_sw[l], c1_sb[l][None, :],
                               c2_lin_w, c2_lin_b[None, :])

    dinv_flat2d = jnp.pad(dinv[:, 0], (0, NPAD - N)).reshape(NROW, 128)
    g_flat = jnp.pad(g[:, 0], (0, NPAD - N))
    for l in range(L):
        aggp2 = _sc_agg2(g_flat, ker2[:, l], src, dst, zeros_flat)
        g2d = _tc_update2(g_flat.reshape(NROW, 128),
                          aggp2.reshape(2, NROW, 128), dinv_flat2d,
                          c2_sw[l], c2_sb[l].reshape(1, 1))
        g_flat = g2d.reshape(NPAD)

    return g_flat[:N, None]


# speedup vs baseline: 7.4279x; 1.0043x over previous
import functools

import jax
import jax.numpy as jnp
from jax import lax
from jax.experimental import pallas as pl
from jax.experimental.pallas import tpu as pltpu
from jax.experimental.pallas import tpu_sc as plsc

N = 10000
E = 320000
IN_CH = 128
HID = 128
OUT_CH = 1
D_EDGE = 16
L = 2

NPAD = 10240
NROW = NPAD // 128
NW = 32
CH = 128
NCHUNK = E // CH
FULL_ROUNDS = NCHUNK // NW
TAIL = NCHUNK - FULL_ROUNDS * NW
CH1 = 64
NCHUNK1 = E // CH1
ROUNDS1 = NCHUNK1 // NW
TAIL1 = NCHUNK1 - ROUNDS1 * NW
NR = 10112
RPT = NR // 16
EPT = E // NW
FPT = NPAD // 16

_BN = 1000


def _sc_mesh():
    return plsc.VectorSubcoreMesh(core_axis_name="c", subcore_axis_name="s",
                                  num_cores=2, num_subcores=16)


def _deg_body(dst_hbm, zeros_hbm, out_hbm, dstb, ones_v, deg_sh, s_dst, s_s):
    c = lax.axis_index("c")
    s = lax.axis_index("s")
    wid = s * 2 + c
    cnt = jnp.where(wid < TAIL, FULL_ROUNDS + 1, FULL_ROUNDS)

    def ob(i, _):
        ones_v[pl.ds(i * 16, 16)] = jnp.full((16,), 1.0, jnp.float32)
        return 0
    lax.fori_loop(0, CH // 16, ob, 0)

    pltpu.sync_copy(zeros_hbm.at[pl.ds(s * FPT, FPT)], deg_sh.at[pl.ds(s * FPT, FPT)])
    plsc.subcore_barrier()

    def e0_of(r):
        return (wid + r * NW) * CH

    def fire_idx(r):
        b4 = lax.rem(r, 4)
        pltpu.async_copy(dst_hbm.at[pl.ds(e0_of(r), CH)], dstb.at[b4], s_dst.at[b4])

    fire_idx(0)
    @pl.when(cnt > 1)
    def _():
        fire_idx(1)

    def body(g, _):
        b2 = lax.rem(g, 2)
        b4 = lax.rem(g, 4)
        pltpu.make_async_copy(dst_hbm.at[pl.ds(0, CH)], dstb.at[b4], s_dst.at[b4]).wait()
        @pl.when(g >= 2)
        def _():
            pltpu.make_async_copy(ones_v, deg_sh.at[dstb.at[b4]], s_s.at[b2]).wait()
        pltpu.async_copy(ones_v, deg_sh.at[dstb.at[b4]], s_s.at[b2], add=True)
        @pl.when(g + 2 < cnt)
        def _():
            fire_idx(g + 2)
        return 0
    lax.fori_loop(0, cnt, body, 0)

    def drain(g):
        b2 = lax.rem(g, 2)
        b4 = lax.rem(g, 4)
        pltpu.make_async_copy(ones_v, deg_sh.at[dstb.at[b4]], s_s.at[b2]).wait()
    drain(cnt - 2)
    drain(cnt - 1)

    plsc.subcore_barrier()
    pltpu.sync_copy(deg_sh.at[pl.ds(s * FPT, FPT)], out_hbm.at[c, pl.ds(s * FPT, FPT)])


def _sc_deg(dst, zeros_flat):
    f = pl.kernel(
        _deg_body,
        out_type=jax.ShapeDtypeStruct((2, NPAD), jnp.float32),
        mesh=_sc_mesh(),
        scratch_types=[
            pltpu.VMEM((4, CH), jnp.int32),
            pltpu.VMEM((CH,), jnp.float32),
            pltpu.VMEM_SHARED((NPAD,), jnp.float32),
            pltpu.SemaphoreType.DMA((4,)),
            pltpu.SemaphoreType.DMA((2,)),
        ],
    )
    return f(dst, zeros_flat)


def _agg_body(h_hbm, ker_hbm, src_hbm, dst_hbm, zeros_hbm, out_hbm,
              srcb, dstb, hs, kerb, msg, agg_sh,
              s_src, s_dst, s_h, s_k, s_s):
    c = lax.axis_index("c")
    s = lax.axis_index("s")
    wid = s * 2 + c
    cnt = jnp.where(wid < TAIL1, ROUNDS1 + 1, ROUNDS1)

    pltpu.sync_copy(zeros_hbm.at[pl.ds(s * RPT, RPT)], agg_sh.at[pl.ds(s * RPT, RPT)])
    plsc.subcore_barrier()

    def e0_of(r):
        return (wid + r * NW) * CH1

    def fire_idx(r):
        b4 = lax.rem(r, 4)
        pltpu.async_copy(src_hbm.at[pl.ds(e0_of(r), CH1)], srcb.at[b4], s_src.at[b4])
        pltpu.async_copy(dst_hbm.at[pl.ds(e0_of(r), CH1)], dstb.at[b4], s_dst.at[b4])

    def fire_ker(r):
        b2 = lax.rem(r, 2)
        pltpu.async_copy(ker_hbm.at[pl.ds(e0_of(r), CH1)], kerb.at[b2], s_k.at[b2])

    def fire_gather(r):
        b4 = lax.rem(r, 4)
        b2 = lax.rem(r, 2)
        pltpu.make_async_copy(src_hbm.at[pl.ds(0, CH1)], srcb.at[b4], s_src.at[b4]).wait()
        pltpu.async_copy(h_hbm.at[srcb.at[b4]], hs.at[b2], s_h.at[b2])

    fire_idx(0)
    fire_ker(0)
    @pl.when(cnt > 1)
    def _():
        fire_idx(1)
        fire_ker(1)
    fire_gather(0)

    def body(g, _):
        b2 = lax.rem(g, 2)
        b4 = lax.rem(g, 4)

        @pl.when(g + 1 < cnt)
        def _():
            fire_gather(g + 1)

        pltpu.make_async_copy(h_hbm.at[srcb.at[b4]], hs.at[b2], s_h.at[b2]).wait()
        pltpu.make_async_copy(ker_hbm.at[pl.ds(0, CH1)], kerb.at[b2], s_k.at[b2]).wait()
        @pl.when(g >= 2)
        def _():
            pltpu.make_async_copy(msg.at[b2], agg_sh.at[dstb.at[b4]], s_s.at[b2]).wait()
        pltpu.make_async_copy(dst_hbm.at[pl.ds(0, CH1)], dstb.at[b4], s_dst.at[b4]).wait()

        @plsc.parallel_loop(0, CH1, unroll=4)
        def _(i):
            for q in range(HID // 32):
                w = kerb[b2, i, pl.ds(q * 16, 16)]
                wh = hs[b2, i, pl.ds(q * 16, 16)]
                ka = lax.bitcast_convert_type(lax.shift_left(w, 16), jnp.float32)
                kb_ = lax.bitcast_convert_type(w & jnp.int32(-65536), jnp.float32)
                ha = lax.bitcast_convert_type(lax.shift_left(wh, 16), jnp.float32)
                hb_ = lax.bitcast_convert_type(wh & jnp.int32(-65536), jnp.float32)
                msg[b2, i, pl.ds(q * 32, 16)] = ha * ka
                msg[b2, i, pl.ds(q * 32 + 16, 16)] = hb_ * kb_

        pltpu.async_copy(msg.at[b2], agg_sh.at[dstb.at[b4]], s_s.at[b2], add=True)

        @pl.when(g + 2 < cnt)
        def _():
            fire_idx(g + 2)
            fire_ker(g + 2)
        return 0
    lax.fori_loop(0, cnt, body, 0)

    def drain(g):
        b2 = lax.rem(g, 2)
        b4 = lax.rem(g, 4)
        pltpu.make_async_copy(msg.at[b2], agg_sh.at[dstb.at[b4]], s_s.at[b2]).wait()
    drain(cnt - 2)
    drain(cnt - 1)

    plsc.subcore_barrier()
    pltpu.sync_copy(agg_sh.at[pl.ds(s * RPT, RPT)], out_hbm.at[c, pl.ds(s * RPT, RPT)])


def _sc_agg(h, ker, src, dst, zeros_h):
    f = pl.kernel(
        _agg_body,
        out_type=jax.ShapeDtypeStruct((2, NR, HID), jnp.float32),
        mesh=_sc_mesh(),
        scratch_types=[
            pltpu.VMEM((4, CH1), jnp.int32),
            pltpu.VMEM((4, CH1), jnp.int32),
            pltpu.VMEM((2, CH1, HID), jnp.int32),
            pltpu.VMEM((2, CH1, HID // 2), jnp.int32),
            pltpu.VMEM((2, CH1, HID), jnp.float32),
            pltpu.VMEM_SHARED((NR, HID), jnp.float32),
            pltpu.SemaphoreType.DMA((4,)),
            pltpu.SemaphoreType.DMA((4,)),
            pltpu.SemaphoreType.DMA((2,)),
            pltpu.SemaphoreType.DMA((2,)),
            pltpu.SemaphoreType.DMA((2,)),
        ],
    )
    return f(h, ker, src, dst, zeros_h)


def _agg2_body(g_hbm, ker_hbm, src_hbm, dst_hbm, zeros_hbm, out_hbm,
               srcb, dstb, hsb, kerb, msgb, agg_sh,
               s_src, s_dst, s_h, s_k, s_s):
    c = lax.axis_index("c")
    s = lax.axis_index("s")
    wid = s * 2 + c
    cnt = jnp.where(wid < TAIL, FULL_ROUNDS + 1, FULL_ROUNDS)

    pltpu.sync_copy(zeros_hbm.at[pl.ds(s * FPT, FPT)], agg_sh.at[pl.ds(s * FPT, FPT)])
    plsc.subcore_barrier()

    def e0_of(r):
        return (wid + r * NW) * CH

    def fire_idx(r):
        b4 = lax.rem(r, 4)
        pltpu.async_copy(src_hbm.at[pl.ds(e0_of(r), CH)], srcb.at[b4], s_src.at[b4])
        pltpu.async_copy(dst_hbm.at[pl.ds(e0_of(r), CH)], dstb.at[b4], s_dst.at[b4])

    def fire_ker(r):
        b2 = lax.rem(r, 2)
        pltpu.async_copy(ker_hbm.at[pl.ds(e0_of(r), CH)], kerb.at[b2], s_k.at[b2])

    def fire_gather(r):
        b4 = lax.rem(r, 4)
        b2 = lax.rem(r, 2)
        pltpu.make_async_copy(src_hbm.at[pl.ds(0, CH)], srcb.at[b4], s_src.at[b4]).wait()
        pltpu.async_copy(g_hbm.at[srcb.at[b4]], hsb.at[b2], s_h.at[b2])

    fire_idx(0)
    fire_ker(0)
    @pl.when(cnt > 1)
    def _():
        fire_idx(1)
        fire_ker(1)
    fire_gather(0)

    def body(g, _):
        b2 = lax.rem(g, 2)
        b4 = lax.rem(g, 4)

        @pl.when(g + 1 < cnt)
        def _():
            fire_gather(g + 1)

        pltpu.make_async_copy(g_hbm.at[srcb.at[b4]], hsb.at[b2], s_h.at[b2]).wait()
        pltpu.make_async_copy(ker_hbm.at[pl.ds(0, CH)], kerb.at[b2], s_k.at[b2]).wait()
        @pl.when(g >= 2)
        def _():
            pltpu.make_async_copy(msgb.at[b2], agg_sh.at[dstb.at[b4]], s_s.at[b2]).wait()
        pltpu.make_async_copy(dst_hbm.at[pl.ds(0, CH)], dstb.at[b4], s_dst.at[b4]).wait()

        @plsc.parallel_loop(0, CH // 16, unroll=4)
        def _(i):
            sl = pl.ds(i * 16, 16)
            msgb[b2, sl] = hsb[b2, sl] * kerb[b2, sl]

        pltpu.async_copy(msgb.at[b2], agg_sh.at[dstb.at[b4]], s_s.at[b2], add=True)

        @pl.when(g + 2 < cnt)
        def _():
            fire_idx(g + 2)
            fire_ker(g + 2)
        return 0
    lax.fori_loop(0, cnt, body, 0)

    def drain(g):
        b2 = lax.rem(g, 2)
        b4 = lax.rem(g, 4)
        pltpu.make_async_copy(msgb.at[b2], agg_sh.at[dstb.at[b4]], s_s.at[b2]).wait()
    drain(cnt - 2)
    drain(cnt - 1)

    plsc.subcore_barrier()
    pltpu.sync_copy(agg_sh.at[pl.ds(s * FPT, FPT)], out_hbm.at[c, pl.ds(s * FPT, FPT)])


def _sc_agg2(g_flat, ker2, src, dst, zeros_flat):
    f = pl.kernel(
        _agg2_body,
        out_type=jax.ShapeDtypeStruct((2, NPAD), jnp.float32),
        mesh=_sc_mesh(),
        scratch_types=[
            pltpu.VMEM((4, CH), jnp.int32),
            pltpu.VMEM((4, CH), jnp.int32),
            pltpu.VMEM((2, CH), jnp.float32),
            pltpu.VMEM((2, CH), jnp.float32),
            pltpu.VMEM((2, CH), jnp.float32),
            pltpu.VMEM_SHARED((NPAD,), jnp.float32),
            pltpu.SemaphoreType.DMA((4,)),
            pltpu.SemaphoreType.DMA((4,)),
            pltpu.SemaphoreType.DMA((2,)),
            pltpu.SemaphoreType.DMA((2,)),
            pltpu.SemaphoreType.DMA((2,)),
        ],
    )
    return f(g_flat, ker2, src, dst, zeros_flat)


def _pack_pairs(x):
    words = []
    for q in range(HID // 32):
        a = lax.bitcast_convert_type(x[:, q * 32:q * 32 + 16], jnp.int32)
        b = lax.bitcast_convert_type(x[:, q * 32 + 16:q * 32 + 32], jnp.int32)
        wa = lax.shift_right_logical(a + 0x8000, 16)
        wb = (b + 0x8000) & jnp.int32(-65536)
        words.append(wa | wb)
    return jnp.concatenate(words, axis=1)


def _ker1_body(ea_ref, kw_ref, kb_ref, o_ref):
    kerf = (jnp.dot(ea_ref[...], kw_ref[...], preferred_element_type=jnp.float32)
            + kb_ref[...])
    o_ref[...] = _pack_pairs(kerf)


def _tc_ker1(edge_attr, kw_l, kb_l):
    be = 4000
    return pl.pallas_call(
        _ker1_body,
        grid=(E // be,),
        in_specs=[
            pl.BlockSpec((be, D_EDGE), lambda i: (i, 0)),
            pl.BlockSpec((D_EDGE, HID), lambda i: (0, 0)),
            pl.BlockSpec((1, HID), lambda i: (0, 0)),
        ],
        out_specs=pl.BlockSpec((be, HID // 2), lambda i: (i, 0)),
        out_shape=jax.ShapeDtypeStruct((E, HID // 2), jnp.int32),
    )(edge_attr, kw_l, kb_l)


def _ker2_body(ea_ref, kw_ref, kb_ref, o_ref):
    o_ref[...] = jnp.dot(ea_ref[...], kw_ref[...], preferred_element_type=jnp.float32) + kb_ref[...]


def _tc_ker2(edge_attr, kw2, kb2):
    be = 8000
    return pl.pallas_call(
        _ker2_body,
        grid=(E // be,),
        in_specs=[
            pl.BlockSpec((be, D_EDGE), lambda i: (i, 0)),
            pl.BlockSpec((D_EDGE, L), lambda i: (0, 0)),
            pl.BlockSpec((1, L), lambda i: (0, 0)),
        ],
        out_specs=pl.BlockSpec((be, L), lambda i: (i, 0)),
        out_shape=jax.ShapeDtypeStruct((E, L), jnp.float32),
    )(edge_attr, kw2, kb2)


def _pe_body(deg_ref, x_ref, w1_ref, b1_ref, w2_ref, b2_ref, lw_ref, lb_ref,
             h_ref, hp_ref, dinv_ref):
    deg = jnp.clip(deg_ref[...], 1.0, None)
    dinv_ref[...] = 1.0 / deg
    pef = jnp.log(1.0 + deg)
    a = jax.nn.relu(pef * w1_ref[...] + b1_ref[...])
    pe = jnp.dot(a, w2_ref[...], preferred_element_type=jnp.float32) + b2_ref[...]
    h0 = x_ref[...] + pe
    h = jnp.dot(h0, lw_ref[...], preferred_element_type=jnp.float32) + lb_ref[...]
    h_ref[...] = h
    p = _pack_pairs(h)
    hp_ref[...] = jnp.concatenate([p, jnp.zeros_like(p)], axis=1)


def _tc_pe(deg_col, x, w1, b1, w2, b2, lw, lb):
    return pl.pallas_call(
        _pe_body,
        grid=(N // _BN,),
        in_specs=[
            pl.BlockSpec((_BN, 1), lambda i: (i, 0)),
            pl.BlockSpec((_BN, IN_CH), lambda i: (i, 0)),
            pl.BlockSpec((1, HID), lambda i: (0, 0)),
            pl.BlockSpec((1, HID), lambda i: (0, 0)),
            pl.BlockSpec((HID, IN_CH), lambda i: (0, 0)),
            pl.BlockSpec((1, IN_CH), lambda i: (0, 0)),
            pl.BlockSpec((IN_CH, HID), lambda i: (0, 0)),
            pl.BlockSpec((1, HID), lambda i: (0, 0)),
        ],
        out_specs=[
            pl.BlockSpec((_BN, HID), lambda i: (i, 0)),
            pl.BlockSpec((_BN, HID), lambda i: (i, 0)),
            pl.BlockSpec((_BN, 1), lambda i: (i, 0)),
        ],
        out_shape=[
            jax.ShapeDtypeStruct((N, HID), jnp.float32),
            jax.ShapeDtypeStruct((N, HID), jnp.int32),
            jax.ShapeDtypeStruct((N, 1), jnp.float32),
        ],
    )(deg_col, x, w1, b1, w2, b2, lw, lb)


def _upd_body(h_ref, aggp_ref, dinv_ref, sw_ref, sb_ref, o_ref, op_ref):
    agg = (aggp_ref[0] + aggp_ref[1]) * dinv_ref[...]
    v = jax.nn.relu(
        jnp.dot(h_ref[...], sw_ref[...], preferred_element_type=jnp.float32)
        + sb_ref[...] + agg)
    o_ref[...] = v
    p = _pack_pairs(v)
    op_ref[...] = jnp.concatenate([p, jnp.zeros_like(p)], axis=1)


def _tc_update(h, aggp, dinv, sw, sb):
    return pl.pallas_call(
        _upd_body,
        grid=(N // _BN,),
        in_specs=[
            pl.BlockSpec((_BN, HID), lambda i: (i, 0)),
            pl.BlockSpec((2, _BN, HID), lambda i: (0, i, 0)),
            pl.BlockSpec((_BN, 1), lambda i: (i, 0)),
            pl.BlockSpec((HID, HID), lambda i: (0, 0)),
            pl.BlockSpec((1, HID), lambda i: (0, 0)),
        ],
        out_specs=[
            pl.BlockSpec((_BN, HID), lambda i: (i, 0)),
            pl.BlockSpec((_BN, HID), lambda i: (i, 0)),
        ],
        out_shape=[
            jax.ShapeDtypeStruct((N, HID), jnp.float32),
            jax.ShapeDtypeStruct((N, HID), jnp.int32),
        ],
    )(h, aggp, dinv, sw, sb)


def _updlin_body(h_ref, aggp_ref, dinv_ref, sw_ref, sb_ref, lw_ref, lb_ref, o_ref):
    agg = (aggp_ref[0] + aggp_ref[1]) * dinv_ref[...]
    h1 = jax.nn.relu(
        jnp.dot(h_ref[...], sw_ref[...], preferred_element_type=jnp.float32)
        + sb_ref[...] + agg)
    o_ref[...] = jnp.dot(h1, lw_ref[...], preferred_element_type=jnp.float32) + lb_ref[...]


def _tc_update_lin(h, aggp, dinv, sw, sb, lw, lb):
    return pl.pallas_call(
        _updlin_body,
        grid=(N // _BN,),
        in_specs=[
            pl.BlockSpec((_BN, HID), lambda i: (i, 0)),
            pl.BlockSpec((2, _BN, HID), lambda i: (0, i, 0)),
            pl.BlockSpec((_BN, 1), lambda i: (i, 0)),
            pl.BlockSpec((HID, HID), lambda i: (0, 0)),
            pl.BlockSpec((1, HID), lambda i: (0, 0)),
            pl.BlockSpec((HID, OUT_CH), lambda i: (0, 0)),
            pl.BlockSpec((1, OUT_CH), lambda i: (0, 0)),
        ],
        out_specs=pl.BlockSpec((_BN, OUT_CH), lambda i: (i, 0)),
        out_shape=jax.ShapeDtypeStruct((N, OUT_CH), jnp.float32),
    )(h, aggp, dinv, sw, sb, lw, lb)


def _upd2_body(g_ref, aggp_ref, dinvf_ref, sw_ref, sb_ref, o_ref):
    agg = jnp.sum(aggp_ref[...], axis=0) * dinvf_ref[...]
    o_ref[...] = jax.nn.relu(g_ref[...] * sw_ref[0, 0] + sb_ref[0, 0] + agg)


def _tc_update2(g_flat2d, aggp, dinv_flat2d, sw_s, sb_s):
    return pl.pallas_call(
        _upd2_body,
        grid=(1,),
        in_specs=[
            pl.BlockSpec((NROW, 128), lambda i: (0, 0)),
            pl.BlockSpec((2, NROW, 128), lambda i: (0, 0, 0)),
            pl.BlockSpec((NROW, 128), lambda i: (0, 0)),
            pl.BlockSpec((1, 1), lambda i: (0, 0)),
            pl.BlockSpec((1, 1), lambda i: (0, 0)),
        ],
        out_specs=pl.BlockSpec((NROW, 128), lambda i: (0, 0)),
        out_shape=jax.ShapeDtypeStruct((NROW, 128), jnp.float32),
    )(g_flat2d, aggp, dinv_flat2d, sw_s, sb_s)


def _red2_body(p_ref, o_ref):
    o_ref[...] = jnp.sum(p_ref[...], axis=0)


def _tc_reduce2(p):
    return pl.pallas_call(
        _red2_body,
        grid=(1,),
        in_specs=[pl.BlockSpec((2, NROW, 128), lambda i: (0, 0, 0))],
        out_specs=pl.BlockSpec((NROW, 128), lambda i: (0, 0)),
        out_shape=jax.ShapeDtypeStruct((NROW, 128), jnp.float32),
    )(p)


def kernel(x, edge_index, edge_attr, pe_w1, pe_b1, pe_w2, pe_b2,
           c1_lin_w, c1_lin_b, c1_kw, c1_kb, c1_sw, c1_sb,
           c2_lin_w, c2_lin_b, c2_kw, c2_kb, c2_sw, c2_sb):
    src = edge_index[0]
    dst = edge_index[1]
    zeros_h = jnp.zeros((NR, HID), jnp.float32)
    zeros_flat = jnp.zeros((NPAD,), jnp.float32)

    degp = _sc_deg(dst, zeros_flat)
    deg = _tc_reduce2(degp.reshape(2, NROW, 128))
    deg_col = deg.reshape(NPAD)[:N, None]

    ker1 = [_tc_ker1(edge_attr, c1_kw[l], c1_kb[l][None, :]) for l in range(L)]
    ker2 = _tc_ker2(edge_attr, jnp.transpose(c2_kw, (1, 2, 0)).reshape(D_EDGE, L),
                    c2_kb.reshape(1, L))

    h, hp, dinv = _tc_pe(deg_col, x, pe_w1, pe_b1[None, :], pe_w2, pe_b2[None, :],
                         c1_lin_w, c1_lin_b[None, :])

    for l in range(L):
        aggp = _sc_agg(hp, ker1[l], src, dst, zeros_h)
        if l < L - 1:
            h, hp = _tc_update(h, aggp, dinv, c1_sw[l], c1_sb[l][None, :])
        else:
            g = _tc_update_lin(h, aggp, dinv, c1_sw[l], c1_sb[l][None, :],
                               c2_lin_w, c2_lin_b[None, :])

    dinv_flat2d = jnp.pad(dinv[:, 0], (0, NPAD - N)).reshape(NROW, 128)
    g_flat = jnp.pad(g[:, 0], (0, NPAD - N))
    for l in range(L):
        aggp2 = _sc_agg2(g_flat, ker2[:, l], src, dst, zeros_flat)
        g2d = _tc_update2(g_flat.reshape(NROW, 128),
                          aggp2.reshape(2, NROW, 128), dinv_flat2d,
                          c2_sw[l], c2_sb[l].reshape(1, 1))
        g_flat = g2d.reshape(NPAD)

    return g_flat[:N, None]
